# Initial kernel scaffold; baseline (speedup 1.0000x reference)
import jax
import jax.numpy as jnp
from jax.experimental import pallas as pl


def kernel(logits, targets):
    raise NotImplementedError("write your pallas kernel here")


# speedup vs baseline: 26.7439x; 26.7439x over previous
import functools

import jax
import jax.numpy as jnp
from jax import lax
from jax.experimental import pallas as pl
from jax.experimental.pallas import tpu as pltpu
from jax.experimental.pallas import tpu_sc as plsc

_C = 19
_B, _H, _W = 8, 512, 512
_N = _B * _H * _W
_K = int(0.2 * _N)
_NC, _NS, _L = 2, 16, 16
_NW = _NC * _NS
_PT = _N // _NW

_BA = 1024
_BB = 2048
_BC = 1024


_BH = 64


def _ce_body(lg_ref, tg_ref, out_f_ref, out_i_ref):
    x = lg_ref[0]
    t = tg_ref[0]
    m = jnp.max(x, axis=0)
    s = jnp.sum(jnp.exp(x - m[None]), axis=0)
    cls = lax.broadcasted_iota(jnp.int32, x.shape, 0)
    xt = jnp.sum(jnp.where(cls == t[None], x, 0.0), axis=0)
    loss = jnp.maximum(m + jnp.log(s) - xt, 0.0)
    out_f_ref[0] = loss
    out_i_ref[0] = lax.bitcast_convert_type(loss, jnp.int32)


def _ce_losses(logits, targets):
    return pl.pallas_call(
        _ce_body,
        grid=(_B, _H // _BH),
        in_specs=[
            pl.BlockSpec((1, _C, _BH, _W), lambda b, h: (b, 0, h, 0)),
            pl.BlockSpec((1, _BH, _W), lambda b, h: (b, h, 0)),
        ],
        out_specs=[
            pl.BlockSpec((1, _BH, _W), lambda b, h: (b, h, 0)),
            pl.BlockSpec((1, _BH, _W), lambda b, h: (b, h, 0)),
        ],
        out_shape=[
            jax.ShapeDtypeStruct((_B, _H, _W), jnp.float32),
            jax.ShapeDtypeStruct((_B, _H, _W), jnp.int32),
        ],
    )(logits, targets)


_MESH = plsc.VectorSubcoreMesh(core_axis_name="c", subcore_axis_name="s")


def _wid():
    return lax.axis_index("s") * _NC + lax.axis_index("c")


def _zero(ref, n, dtype):
    z = jnp.zeros((_L,), dtype)

    def body(j, _):
        ref[pl.ds(j * _L, _L)] = z
        return 0

    lax.fori_loop(0, n // _L, body, 0)


def _reduce_rows(hist_hbm, buf_v, acc_v, nbins):
    rows = 8
    _zero(acc_v, nbins, jnp.int32)

    def chunk(ci, _):
        src = hist_hbm.at[pl.ds(pl.multiple_of(ci * (rows * nbins), 8),
                                rows * nbins)]
        pltpu.sync_copy(src, buf_v.at[pl.ds(0, rows * nbins)])

        def jbody(j, __):
            acc = acc_v[pl.ds(j * _L, _L)]
            for rr in range(rows):
                acc = acc + buf_v[pl.ds(rr * nbins + j * _L, _L)]
            acc_v[pl.ds(j * _L, _L)] = acc
            return 0

        lax.fori_loop(0, nbins // _L, jbody, 0)
        return 0

    lax.fori_loop(0, _NW // rows, chunk, 0)


def _find_kth(acc_v, nbins, kk):
    nch = nbins // _L
    lane = lax.iota(jnp.int32, _L)

    def body(i, carry):
        found, bfound, above, cum = carry
        j = nch - 1 - i
        v = acc_v[pl.ds(j * _L, _L)]
        rv = lax.rev(v, (0,))
        cs = plsc.cumsum(rv)
        tot = jnp.max(cs)
        hit = (cum + cs) >= kk
        anyhit = jnp.max(hit.astype(jnp.int32)) > 0
        ps = jnp.max(plsc.all_reduce_ffs(hit))
        bin_here = j * _L + (_L - 1) - ps
        above_here = cum + jnp.sum(jnp.where(lane < ps, rv, 0))
        take = jnp.logical_and(anyhit, found == 0)
        return (jnp.where(take, 1, found),
                jnp.where(take, bin_here, bfound),
                jnp.where(take, above_here, above),
                cum + tot)

    _, b, above, _ = lax.fori_loop(0, nch, body, (0, 0, 0, 0))
    return b, above


def _lane_reduce_store(tbl_v, red_v, nbins):
    def rbody(j, _):
        acc = tbl_v[pl.ds(j * _L, _L)]
        for l in range(1, _L):
            acc = acc + tbl_v[pl.ds(l * nbins + j * _L, _L)]
        red_v[pl.ds(j * _L, _L)] = acc
        return 0

    lax.fori_loop(0, nbins // _L, rbody, 0)


@functools.partial(
    pl.kernel,
    out_type=jax.ShapeDtypeStruct((_NW * _BA,), jnp.int32),
    mesh=_MESH,
    compiler_params=pltpu.CompilerParams(needs_layout_passes=False),
    scratch_types=[
        pltpu.VMEM((_PT,), jnp.int32),
        pltpu.VMEM((_L * _BA,), jnp.int32),
        pltpu.VMEM((_BA,), jnp.int32),
    ],
)
def _hist_a(bits_hbm, out_hbm, data_v, tbl_v, red_v):
    wid = _wid()
    pltpu.sync_copy(bits_hbm.at[pl.ds(pl.multiple_of(wid * _PT, 8), _PT)],
                    data_v)
    _zero(tbl_v, _L * _BA, jnp.int32)
    lb = lax.iota(jnp.int32, _L) * _BA
    ones = jnp.ones((_L,), jnp.int32)

    def body(i, _):
        bits = data_v[pl.ds(i * _L, _L)]
        plsc.addupdate_scatter(tbl_v, [lb + (bits >> 21)], ones)
        return 0

    lax.fori_loop(0, _PT // _L, body, 0)
    _lane_reduce_store(tbl_v, red_v, _BA)
    pltpu.sync_copy(red_v,
                    out_hbm.at[pl.ds(pl.multiple_of(wid * _BA, 8), _BA)])


@functools.partial(
    pl.kernel,
    out_type=jax.ShapeDtypeStruct((_NW * _BB,), jnp.int32),
    mesh=_MESH,
    compiler_params=pltpu.CompilerParams(needs_layout_passes=False),
    scratch_types=[
        pltpu.VMEM((_PT,), jnp.int32),
        pltpu.VMEM((_L * _BB,), jnp.int32),
        pltpu.VMEM((_BB,), jnp.int32),
        pltpu.VMEM((8 * _BA,), jnp.int32),
        pltpu.VMEM((_BA,), jnp.int32),
    ],
)
def _hist_b(bits_hbm, ha_hbm, out_hbm, data_v, tbl_v, red_v, buf_v, acca_v):
    wid = _wid()
    _reduce_rows(ha_hbm, buf_v, acca_v, _BA)
    ba, _ = _find_kth(acca_v, _BA, _K)
    pltpu.sync_copy(bits_hbm.at[pl.ds(pl.multiple_of(wid * _PT, 8), _PT)],
                    data_v)
    _zero(tbl_v, _L * _BB, jnp.int32)
    lb = lax.iota(jnp.int32, _L) * _BB
    ones = jnp.ones((_L,), jnp.int32)

    def body(i, _):
        bits = data_v[pl.ds(i * _L, _L)]
        m = (bits >> 21) == ba
        plsc.addupdate_scatter(tbl_v, [lb + ((bits >> 10) & 0x7FF)], ones,
                               mask=m)
        return 0

    lax.fori_loop(0, _PT // _L, body, 0)
    _lane_reduce_store(tbl_v, red_v, _BB)
    pltpu.sync_copy(red_v,
                    out_hbm.at[pl.ds(pl.multiple_of(wid * _BB, 8), _BB)])


@functools.partial(
    pl.kernel,
    out_type=jax.ShapeDtypeStruct((_NW * _BC,), jnp.int32),
    mesh=_MESH,
    compiler_params=pltpu.CompilerParams(needs_layout_passes=False),
    scratch_types=[
        pltpu.VMEM((_PT,), jnp.int32),
        pltpu.VMEM((_L * _BC,), jnp.int32),
        pltpu.VMEM((_BC,), jnp.int32),
        pltpu.VMEM((8 * _BB,), jnp.int32),
        pltpu.VMEM((_BA,), jnp.int32),
        pltpu.VMEM((_BB,), jnp.int32),
    ],
)
def _hist_c(bits_hbm, ha_hbm, hb_hbm, out_hbm, data_v, tbl_v, red_v, buf_v,
            acca_v, accb_v):
    wid = _wid()
    _reduce_rows(ha_hbm, buf_v, acca_v, _BA)
    ba, above_a = _find_kth(acca_v, _BA, _K)
    _reduce_rows(hb_hbm, buf_v, accb_v, _BB)
    bb, _ = _find_kth(accb_v, _BB, _K - above_a)
    pfx = ba * _BB + bb
    pltpu.sync_copy(bits_hbm.at[pl.ds(pl.multiple_of(wid * _PT, 8), _PT)],
                    data_v)
    _zero(tbl_v, _L * _BC, jnp.int32)
    lb = lax.iota(jnp.int32, _L) * _BC
    ones = jnp.ones((_L,), jnp.int32)

    def body(i, _):
        bits = data_v[pl.ds(i * _L, _L)]
        m = (bits >> 10) == pfx
        plsc.addupdate_scatter(tbl_v, [lb + (bits & 0x3FF)], ones, mask=m)
        return 0

    lax.fori_loop(0, _PT // _L, body, 0)
    _lane_reduce_store(tbl_v, red_v, _BC)
    pltpu.sync_copy(red_v,
                    out_hbm.at[pl.ds(pl.multiple_of(wid * _BC, 8), _BC)])


_CH = 8192


@functools.partial(
    pl.kernel,
    out_type=(jax.ShapeDtypeStruct((_NW * 128,), jnp.float32),
              jax.ShapeDtypeStruct((_NW * _L,), jnp.int32)),
    mesh=_MESH,
    compiler_params=pltpu.CompilerParams(needs_layout_passes=False),
    scratch_types=[
        pltpu.VMEM((_CH,), jnp.float32),
        pltpu.VMEM((_CH,), jnp.int32),
        pltpu.VMEM((_CH,), jnp.int32),
        pltpu.VMEM((3 * _L * 32,), jnp.float32),
        pltpu.VMEM((128,), jnp.float32),
        pltpu.VMEM((8 * _BB,), jnp.int32),
        pltpu.VMEM((_BA,), jnp.int32),
        pltpu.VMEM((_BB,), jnp.int32),
        pltpu.VMEM((_BC,), jnp.int32),
    ],
)
def _stats(loss_hbm, bits_hbm, tgt_hbm, ha_hbm, hb_hbm, hc_hbm, out_hbm,
           meta_hbm, dataf_v, datai_v, tgt_v, tbl_v, stg_v, buf_v, acca_v,
           accb_v, accc_v):
    wid = _wid()
    _reduce_rows(ha_hbm, buf_v, acca_v, _BA)
    ba, above_a = _find_kth(acca_v, _BA, _K)
    _reduce_rows(hb_hbm, buf_v, accb_v, _BB)
    bb, above_b = _find_kth(accb_v, _BB, _K - above_a)
    _reduce_rows(hc_hbm, buf_v, accc_v, _BC)
    bc, _ = _find_kth(accc_v, _BC, _K - above_a - above_b)
    tau_bits = (ba << 21) | (bb << 10) | bc

    _zero(tbl_v, 3 * _L * 32, jnp.float32)
    lane32 = lax.iota(jnp.int32, _L) * 32
    onesf = jnp.ones((_L,), jnp.float32)

    def tchunk(c, _):
        base = pl.multiple_of(wid * _PT + c * _CH, 8)
        pltpu.sync_copy(loss_hbm.at[pl.ds(base, _CH)], dataf_v)
        pltpu.sync_copy(bits_hbm.at[pl.ds(base, _CH)], datai_v)
        pltpu.sync_copy(tgt_hbm.at[pl.ds(base, _CH)], tgt_v)

        def body(i, __):
            v = dataf_v[pl.ds(i * _L, _L)]
            bits = datai_v[pl.ds(i * _L, _L)]
            tg = tgt_v[pl.ds(i * _L, _L)]
            mg = bits > tau_bits
            me = bits == tau_bits
            idx = lane32 + tg
            plsc.addupdate_scatter(tbl_v, [idx], onesf, mask=mg)
            plsc.addupdate_scatter(tbl_v, [idx + _L * 32], v, mask=mg)
            plsc.addupdate_scatter(tbl_v, [idx + 2 * _L * 32], onesf, mask=me)
            return 0

        lax.fori_loop(0, _CH // _L, body, 0)
        return 0

    lax.fori_loop(0, _PT // _CH, tchunk, 0)

    for r in range(3):
        def rbody(j, _, r=r):
            acc = tbl_v[pl.ds(r * _L * 32 + j * _L, _L)]
            for l in range(1, _L):
                acc = acc + tbl_v[pl.ds(r * _L * 32 + l * 32 + j * _L, _L)]
            stg_v[pl.ds(r * 32 + j * _L, _L)] = acc
            return 0

        lax.fori_loop(0, 2, rbody, 0)
    stg_v[pl.ds(96, _L)] = jnp.zeros((_L,), jnp.float32)
    stg_v[pl.ds(112, _L)] = jnp.zeros((_L,), jnp.float32)
    pltpu.sync_copy(stg_v,
                    out_hbm.at[pl.ds(pl.multiple_of(wid * 128, 8), 128)])
    tgt_v[pl.ds(0, _L)] = jnp.full((_L,), tau_bits, jnp.int32)
    pltpu.sync_copy(tgt_v.at[pl.ds(0, _L)],
                    meta_hbm.at[pl.ds(pl.multiple_of(wid * _L, 8), _L)])


def _comb_body(st_ref, tau_ref, out_ref):
    x = st_ref[...]
    cnt = jnp.sum(x[:, 0:32], axis=0)
    s = jnp.sum(x[:, 32:64], axis=0)
    t = jnp.sum(x[:, 64:96], axis=0)
    tau = tau_ref[0, 0]
    r = _K - jnp.sum(cnt)
    ii = lax.broadcasted_iota(jnp.int32, (32, 32), 0)
    jj = lax.broadcasted_iota(jnp.int32, (32, 32), 1)
    pre = jnp.sum(jnp.where(ii < jj, t[:, None], 0.0), axis=0)
    a = jnp.clip(r - pre, 0.0, t)
    cnt_tot = cnt + a
    s_tot = s + a * tau
    contrib = s_tot * lax.rsqrt(cnt_tot + 1e-8)
    out_ref[...] = jnp.reshape(jnp.sum(contrib) * (1.0 / _K ** 0.5), (1, 1))


def _combine(stats, tau):
    return pl.pallas_call(
        _comb_body,
        in_specs=[
            pl.BlockSpec((_NW, 128), lambda: (0, 0)),
            pl.BlockSpec(memory_space=pltpu.SMEM),
        ],
        out_shape=jax.ShapeDtypeStruct((1, 1), jnp.float32),
    )(stats, tau)


def kernel(logits, targets):
    loss_f, loss_i = _ce_losses(logits, targets)
    loss_f = loss_f.reshape(_N)
    loss_i = loss_i.reshape(_N)
    tflat = targets.reshape(_N)
    ha = _hist_a(loss_i)
    hb = _hist_b(loss_i, ha)
    hc = _hist_c(loss_i, ha, hb)
    st, meta = _stats(loss_f, loss_i, tflat, ha, hb, hc)
    tau = lax.bitcast_convert_type(meta[:1], jnp.float32).reshape(1, 1)
    return _combine(st.reshape(_NW, 128), tau).reshape(())


# speedup vs baseline: 35.4694x; 1.3263x over previous
import functools

import jax
import jax.numpy as jnp
from jax import lax
from jax.experimental import pallas as pl
from jax.experimental.pallas import tpu as pltpu
from jax.experimental.pallas import tpu_sc as plsc

_C = 19
_B, _H, _W = 8, 512, 512
_N = _B * _H * _W
_K = int(0.2 * _N)
_NC, _NS, _L = 2, 16, 16
_NW = _NC * _NS
_PT = _N // _NW

_BA = 1024
_BB = 2048
_BC = 1024


_BH = 64


def _ce_body(lg_ref, tg_ref, out_f_ref, out_i_ref):
    x = lg_ref[0]
    t = tg_ref[0]
    m = jnp.max(x, axis=0)
    s = jnp.sum(jnp.exp(x - m[None]), axis=0)
    cls = lax.broadcasted_iota(jnp.int32, x.shape, 0)
    xt = jnp.sum(jnp.where(cls == t[None], x, 0.0), axis=0)
    loss = jnp.maximum(m + jnp.log(s) - xt, 0.0)
    out_f_ref[0] = loss
    out_i_ref[0] = lax.bitcast_convert_type(loss, jnp.int32)


def _ce_losses(logits, targets):
    return pl.pallas_call(
        _ce_body,
        grid=(_B, _H // _BH),
        in_specs=[
            pl.BlockSpec((1, _C, _BH, _W), lambda b, h: (b, 0, h, 0)),
            pl.BlockSpec((1, _BH, _W), lambda b, h: (b, h, 0)),
        ],
        out_specs=[
            pl.BlockSpec((1, _BH, _W), lambda b, h: (b, h, 0)),
            pl.BlockSpec((1, _BH, _W), lambda b, h: (b, h, 0)),
        ],
        out_shape=[
            jax.ShapeDtypeStruct((_B, _H, _W), jnp.float32),
            jax.ShapeDtypeStruct((_B, _H, _W), jnp.int32),
        ],
    )(logits, targets)


_MESH = plsc.VectorSubcoreMesh(core_axis_name="c", subcore_axis_name="s")


def _wid():
    return lax.axis_index("s") * _NC + lax.axis_index("c")


def _zero(ref, n, dtype):
    z = jnp.zeros((_L,), dtype)

    def body(j, _):
        ref[pl.ds(j * _L, _L)] = z
        return 0

    lax.fori_loop(0, n // _L, body, 0)


def _reduce_rows(hist_hbm, buf_v, acc_v, nbins):
    rows = 8
    _zero(acc_v, nbins, jnp.int32)

    def chunk(ci, _):
        src = hist_hbm.at[pl.ds(pl.multiple_of(ci * (rows * nbins), 8),
                                rows * nbins)]
        pltpu.sync_copy(src, buf_v.at[pl.ds(0, rows * nbins)])

        def jbody(j, __):
            acc = acc_v[pl.ds(j * _L, _L)]
            for rr in range(rows):
                acc = acc + buf_v[pl.ds(rr * nbins + j * _L, _L)]
            acc_v[pl.ds(j * _L, _L)] = acc
            return 0

        lax.fori_loop(0, nbins // _L, jbody, 0)
        return 0

    lax.fori_loop(0, _NW // rows, chunk, 0)


def _find_kth(acc_v, nbins, kk):
    nch = nbins // _L
    lane = lax.iota(jnp.int32, _L)

    def body(i, carry):
        found, bfound, above, cum = carry
        j = nch - 1 - i
        v = acc_v[pl.ds(j * _L, _L)]
        rv = lax.rev(v, (0,))
        cs = plsc.cumsum(rv)
        tot = jnp.max(cs)
        hit = (cum + cs) >= kk
        anyhit = jnp.max(hit.astype(jnp.int32)) > 0
        ps = jnp.max(plsc.all_reduce_ffs(hit))
        bin_here = j * _L + (_L - 1) - ps
        above_here = cum + jnp.sum(jnp.where(lane < ps, rv, 0))
        take = jnp.logical_and(anyhit, found == 0)
        return (jnp.where(take, 1, found),
                jnp.where(take, bin_here, bfound),
                jnp.where(take, above_here, above),
                cum + tot)

    _, b, above, _ = lax.fori_loop(0, nch, body, (0, 0, 0, 0))
    return b, above


def _lane_reduce_store(tbl_v, red_v, nbins):
    def rbody(j, _):
        acc = tbl_v[pl.ds(j * _L, _L)]
        for l in range(1, _L):
            acc = acc + tbl_v[pl.ds(l * nbins + j * _L, _L)]
        red_v[pl.ds(j * _L, _L)] = acc
        return 0

    lax.fori_loop(0, nbins // _L, rbody, 0)


@functools.partial(
    pl.kernel,
    out_type=jax.ShapeDtypeStruct((_NW * _BA,), jnp.int32),
    mesh=_MESH,
    compiler_params=pltpu.CompilerParams(needs_layout_passes=False),
    scratch_types=[
        pltpu.VMEM((_PT,), jnp.int32),
        pltpu.VMEM((_L * _BA,), jnp.int32),
        pltpu.VMEM((_BA,), jnp.int32),
    ],
)
def _hist_a(bits_hbm, out_hbm, data_v, tbl_v, red_v):
    wid = _wid()
    pltpu.sync_copy(bits_hbm.at[pl.ds(pl.multiple_of(wid * _PT, 8), _PT)],
                    data_v)
    _zero(tbl_v, _L * _BA, jnp.int32)
    lb = lax.iota(jnp.int32, _L) * _BA
    ones = jnp.ones((_L,), jnp.int32)

    def body(i, _):
        bits = data_v[pl.ds(i * _L, _L)]
        plsc.addupdate_scatter(tbl_v, [lb + (bits >> 21)], ones)
        return 0

    lax.fori_loop(0, _PT // _L, body, 0)
    _lane_reduce_store(tbl_v, red_v, _BA)
    pltpu.sync_copy(red_v,
                    out_hbm.at[pl.ds(pl.multiple_of(wid * _BA, 8), _BA)])


@functools.partial(
    pl.kernel,
    out_type=jax.ShapeDtypeStruct((_NW * _BB,), jnp.int32),
    mesh=_MESH,
    compiler_params=pltpu.CompilerParams(needs_layout_passes=False),
    scratch_types=[
        pltpu.VMEM((_PT,), jnp.int32),
        pltpu.VMEM((_L * _BB,), jnp.int32),
        pltpu.VMEM((_BB,), jnp.int32),
        pltpu.VMEM((8 * _BA,), jnp.int32),
        pltpu.VMEM((_BA,), jnp.int32),
    ],
)
def _hist_b(bits_hbm, ha_hbm, out_hbm, data_v, tbl_v, red_v, buf_v, acca_v):
    wid = _wid()
    _reduce_rows(ha_hbm, buf_v, acca_v, _BA)
    ba, _ = _find_kth(acca_v, _BA, _K)
    pltpu.sync_copy(bits_hbm.at[pl.ds(pl.multiple_of(wid * _PT, 8), _PT)],
                    data_v)
    _zero(tbl_v, _L * _BB, jnp.int32)
    lb = lax.iota(jnp.int32, _L) * _BB
    ones = jnp.ones((_L,), jnp.int32)

    def body(i, _):
        bits = data_v[pl.ds(i * _L, _L)]
        m = (bits >> 21) == ba
        plsc.addupdate_scatter(tbl_v, [lb + ((bits >> 10) & 0x7FF)], ones,
                               mask=m)
        return 0

    lax.fori_loop(0, _PT // _L, body, 0)
    _lane_reduce_store(tbl_v, red_v, _BB)
    pltpu.sync_copy(red_v,
                    out_hbm.at[pl.ds(pl.multiple_of(wid * _BB, 8), _BB)])


_CH = 16384


@functools.partial(
    pl.kernel,
    out_type=jax.ShapeDtypeStruct((_NW * 128,), jnp.float32),
    mesh=_MESH,
    compiler_params=pltpu.CompilerParams(needs_layout_passes=False),
    scratch_types=[
        pltpu.VMEM((_CH,), jnp.int32),
        pltpu.VMEM((_CH,), jnp.int32),
        pltpu.VMEM((_CH,), jnp.float32),
        pltpu.VMEM((_CH,), jnp.float32),
        pltpu.VMEM((_CH,), jnp.int32),
        pltpu.VMEM((_CH,), jnp.int32),
        pltpu.VMEM((4 * _L * 32,), jnp.float32),
        pltpu.VMEM((128,), jnp.float32),
        pltpu.VMEM((8 * _BB,), jnp.int32),
        pltpu.VMEM((_BA,), jnp.int32),
        pltpu.VMEM((_BB,), jnp.int32),
        pltpu.SemaphoreType.DMA,
    ],
)
def _stats(loss_hbm, bits_hbm, tgt_hbm, ha_hbm, hb_hbm, out_hbm,
           bi0_v, bi1_v, lo0_v, lo1_v, tg0_v, tg1_v, tbl_v, stg_v, buf_v,
           acca_v, accb_v, sem):
    wid = _wid()
    nch = _PT // _CH
    bi = (bi0_v, bi1_v)
    lo = (lo0_v, lo1_v)
    tg = (tg0_v, tg1_v)

    def start(c, b):
        base = pl.multiple_of(wid * _PT + c * _CH, 8)
        return (pltpu.async_copy(bits_hbm.at[pl.ds(base, _CH)], bi[b], sem),
                pltpu.async_copy(loss_hbm.at[pl.ds(base, _CH)], lo[b], sem),
                pltpu.async_copy(tgt_hbm.at[pl.ds(base, _CH)], tg[b], sem))

    hs = start(0, 0)
    _reduce_rows(ha_hbm, buf_v, acca_v, _BA)
    ba, above_a = _find_kth(acca_v, _BA, _K)
    _reduce_rows(hb_hbm, buf_v, accb_v, _BB)
    bb, _ = _find_kth(accb_v, _BB, _K - above_a)
    pfx = ba * _BB + bb

    _zero(tbl_v, 4 * _L * 32, jnp.float32)
    lane32 = lax.iota(jnp.int32, _L) * 32
    onesf = jnp.ones((_L,), jnp.float32)

    for c in range(nch):
        for h in hs:
            h.wait()
        if c + 1 < nch:
            hs = start(c + 1, (c + 1) % 2)
        bb_v, ll_v, tt_v = bi[c % 2], lo[c % 2], tg[c % 2]

        def body(i, _, bb_v=bb_v, ll_v=ll_v, tt_v=tt_v):
            bits = bb_v[pl.ds(i * _L, _L)]
            v = ll_v[pl.ds(i * _L, _L)]
            t = tt_v[pl.ds(i * _L, _L)]
            hi = bits >> 10
            mg = hi > pfx
            ma = hi >= pfx
            idx = lane32 + t
            off1 = jnp.where(mg, 0, 3 * _L * 32)
            off2 = jnp.where(mg, _L * 32, 2 * _L * 32)
            plsc.addupdate_scatter(tbl_v, [idx + off1], onesf, mask=ma)
            plsc.addupdate_scatter(tbl_v, [idx + off2], v, mask=ma)
            return 0

        lax.fori_loop(0, _CH // _L, body, 0)

    for r, so in ((0, 0), (1, 32), (3, 64), (2, 96)):
        def rbody(j, _, r=r, so=so):
            acc = tbl_v[pl.ds(r * _L * 32 + j * _L, _L)]
            for l in range(1, _L):
                acc = acc + tbl_v[pl.ds(r * _L * 32 + l * 32 + j * _L, _L)]
            stg_v[pl.ds(so + j * _L, _L)] = acc
            return 0

        lax.fori_loop(0, 2, rbody, 0)
    pltpu.sync_copy(stg_v,
                    out_hbm.at[pl.ds(pl.multiple_of(wid * 128, 8), 128)])


def _comb_body(st_ref, out_ref):
    x = st_ref[...]
    cnt = jnp.sum(x[:, 0:32], axis=0)
    s = jnp.sum(x[:, 32:64], axis=0)
    t = jnp.sum(x[:, 64:96], axis=0)
    stie = jnp.sum(x[:, 96:128], axis=0)
    r = _K - jnp.sum(cnt)
    ii = lax.broadcasted_iota(jnp.int32, (32, 32), 0)
    jj = lax.broadcasted_iota(jnp.int32, (32, 32), 1)
    pre = jnp.sum(jnp.where(ii < jj, t[:, None], 0.0), axis=0)
    a = jnp.clip(r - pre, 0.0, t)
    cnt_tot = cnt + a
    s_tot = s + a * (stie / jnp.maximum(t, 1.0))
    contrib = s_tot * lax.rsqrt(cnt_tot + 1e-8)
    out_ref[...] = jnp.reshape(jnp.sum(contrib) * (1.0 / _K ** 0.5), (1, 1))


def _combine(stats):
    return pl.pallas_call(
        _comb_body,
        in_specs=[pl.BlockSpec((_NW, 128), lambda: (0, 0))],
        out_shape=jax.ShapeDtypeStruct((1, 1), jnp.float32),
    )(stats)


def kernel(logits, targets):
    loss_f, loss_i = _ce_losses(logits, targets)
    loss_f = loss_f.reshape(_N)
    loss_i = loss_i.reshape(_N)
    tflat = targets.reshape(_N)
    ha = _hist_a(loss_i)
    hb = _hist_b(loss_i, ha)
    st = _stats(loss_f, loss_i, tflat, ha, hb)
    return _combine(st.reshape(_NW, 128)).reshape(())


# speedup vs baseline: 39.2962x; 1.1079x over previous
import functools

import jax
import jax.numpy as jnp
from jax import lax
from jax.experimental import pallas as pl
from jax.experimental.pallas import tpu as pltpu
from jax.experimental.pallas import tpu_sc as plsc

_C = 19
_B, _H, _W = 8, 512, 512
_N = _B * _H * _W
_K = int(0.2 * _N)
_NC, _NS, _L = 2, 16, 16
_NW = _NC * _NS
_PT = _N // _NW

_BA = 1024
_BB = 2048
_BC = 1024


_BH = 64


def _ce_body(lg_ref, tg_ref, out_f_ref, out_i_ref, out_t_ref):
    x = lg_ref[0]
    t = tg_ref[0]
    m = jnp.max(x, axis=0)
    s = jnp.sum(jnp.exp(x - m[None]), axis=0)
    cls = lax.broadcasted_iota(jnp.int32, x.shape, 0)
    xt = jnp.sum(jnp.where(cls == t[None], x, 0.0), axis=0)
    loss = jnp.maximum(m + jnp.log(s) - xt, 0.0)
    out_f_ref[...] = loss.reshape(_BH * _W)
    out_i_ref[...] = lax.bitcast_convert_type(loss, jnp.int32).reshape(_BH * _W)
    out_t_ref[...] = t.reshape(_BH * _W)


def _ce_losses(logits, targets):
    flat = pl.BlockSpec((_BH * _W,), lambda b, h: (b * (_H // _BH) + h,))
    return pl.pallas_call(
        _ce_body,
        grid=(_B, _H // _BH),
        in_specs=[
            pl.BlockSpec((1, _C, _BH, _W), lambda b, h: (b, 0, h, 0)),
            pl.BlockSpec((1, _BH, _W), lambda b, h: (b, h, 0)),
        ],
        out_specs=[flat, flat, flat],
        out_shape=[
            jax.ShapeDtypeStruct((_N,), jnp.float32),
            jax.ShapeDtypeStruct((_N,), jnp.int32),
            jax.ShapeDtypeStruct((_N,), jnp.int32),
        ],
    )(logits, targets)


_MESH = plsc.VectorSubcoreMesh(core_axis_name="c", subcore_axis_name="s")


def _wid():
    return lax.axis_index("s") * _NC + lax.axis_index("c")


def _zero(ref, n, dtype):
    z = jnp.zeros((_L,), dtype)

    def body(j, _):
        ref[pl.ds(j * _L, _L)] = z
        return 0

    lax.fori_loop(0, n // _L, body, 0)


def _reduce_rows(hist_hbm, buf_v, acc_v, nbins):
    rows = 8
    _zero(acc_v, nbins, jnp.int32)

    def chunk(ci, _):
        src = hist_hbm.at[pl.ds(pl.multiple_of(ci * (rows * nbins), 8),
                                rows * nbins)]
        pltpu.sync_copy(src, buf_v.at[pl.ds(0, rows * nbins)])

        def jbody(j, __):
            acc = acc_v[pl.ds(j * _L, _L)]
            for rr in range(rows):
                acc = acc + buf_v[pl.ds(rr * nbins + j * _L, _L)]
            acc_v[pl.ds(j * _L, _L)] = acc
            return 0

        lax.fori_loop(0, nbins // _L, jbody, 0)
        return 0

    lax.fori_loop(0, _NW // rows, chunk, 0)


def _find_kth(acc_v, nbins, kk):
    nch = nbins // _L
    lane = lax.iota(jnp.int32, _L)

    def body(i, carry):
        found, bfound, above, cum = carry
        j = nch - 1 - i
        v = acc_v[pl.ds(j * _L, _L)]
        rv = lax.rev(v, (0,))
        cs = plsc.cumsum(rv)
        tot = jnp.max(cs)
        hit = (cum + cs) >= kk
        anyhit = jnp.max(hit.astype(jnp.int32)) > 0
        ps = jnp.max(plsc.all_reduce_ffs(hit))
        bin_here = j * _L + (_L - 1) - ps
        above_here = cum + jnp.sum(jnp.where(lane < ps, rv, 0))
        take = jnp.logical_and(anyhit, found == 0)
        return (jnp.where(take, 1, found),
                jnp.where(take, bin_here, bfound),
                jnp.where(take, above_here, above),
                cum + tot)

    _, b, above, _ = lax.fori_loop(0, nch, body, (0, 0, 0, 0))
    return b, above


def _lane_reduce_store(tbl_v, red_v, nbins):
    def rbody(j, _):
        acc = tbl_v[pl.ds(j * _L, _L)]
        for l in range(1, _L):
            acc = acc + tbl_v[pl.ds(l * nbins + j * _L, _L)]
        red_v[pl.ds(j * _L, _L)] = acc
        return 0

    lax.fori_loop(0, nbins // _L, rbody, 0)


@functools.partial(
    pl.kernel,
    out_type=jax.ShapeDtypeStruct((_NW * _BA,), jnp.int32),
    mesh=_MESH,
    compiler_params=pltpu.CompilerParams(needs_layout_passes=False),
    scratch_types=[
        pltpu.VMEM((_PT,), jnp.int32),
        pltpu.VMEM((_L * _BA,), jnp.int32),
        pltpu.VMEM((_BA,), jnp.int32),
    ],
)
def _hist_a(bits_hbm, out_hbm, data_v, tbl_v, red_v):
    wid = _wid()
    pltpu.sync_copy(bits_hbm.at[pl.ds(pl.multiple_of(wid * _PT, 8), _PT)],
                    data_v)
    _zero(tbl_v, _L * _BA, jnp.int32)
    lb = lax.iota(jnp.int32, _L) * _BA
    ones = jnp.ones((_L,), jnp.int32)

    def body(i, _):
        bits = data_v[pl.ds(i * _L, _L)]
        plsc.addupdate_scatter(tbl_v, [lb + (bits >> 21)], ones)
        return 0

    lax.fori_loop(0, _PT // _L, body, 0)
    _lane_reduce_store(tbl_v, red_v, _BA)
    pltpu.sync_copy(red_v,
                    out_hbm.at[pl.ds(pl.multiple_of(wid * _BA, 8), _BA)])


@functools.partial(
    pl.kernel,
    out_type=jax.ShapeDtypeStruct((_NW * _BB,), jnp.int32),
    mesh=_MESH,
    compiler_params=pltpu.CompilerParams(needs_layout_passes=False),
    scratch_types=[
        pltpu.VMEM((_PT,), jnp.int32),
        pltpu.VMEM((_L * _BB,), jnp.int32),
        pltpu.VMEM((_BB,), jnp.int32),
        pltpu.VMEM((8 * _BA,), jnp.int32),
        pltpu.VMEM((_BA,), jnp.int32),
    ],
)
def _hist_b(bits_hbm, ha_hbm, out_hbm, data_v, tbl_v, red_v, buf_v, acca_v):
    wid = _wid()
    _reduce_rows(ha_hbm, buf_v, acca_v, _BA)
    ba, _ = _find_kth(acca_v, _BA, _K)
    pltpu.sync_copy(bits_hbm.at[pl.ds(pl.multiple_of(wid * _PT, 8), _PT)],
                    data_v)
    _zero(tbl_v, _L * _BB, jnp.int32)
    lb = lax.iota(jnp.int32, _L) * _BB
    ones = jnp.ones((_L,), jnp.int32)

    def body(i, _):
        bits = data_v[pl.ds(i * _L, _L)]
        m = (bits >> 21) == ba
        plsc.addupdate_scatter(tbl_v, [lb + ((bits >> 10) & 0x7FF)], ones,
                               mask=m)
        return 0

    lax.fori_loop(0, _PT // _L, body, 0)
    _lane_reduce_store(tbl_v, red_v, _BB)
    pltpu.sync_copy(red_v,
                    out_hbm.at[pl.ds(pl.multiple_of(wid * _BB, 8), _BB)])


_CH = 16384


@functools.partial(
    pl.kernel,
    out_type=jax.ShapeDtypeStruct((_NW * 128,), jnp.float32),
    mesh=_MESH,
    compiler_params=pltpu.CompilerParams(needs_layout_passes=False),
    scratch_types=[
        pltpu.VMEM((_CH,), jnp.int32),
        pltpu.VMEM((_CH,), jnp.int32),
        pltpu.VMEM((_CH,), jnp.float32),
        pltpu.VMEM((_CH,), jnp.float32),
        pltpu.VMEM((_CH,), jnp.int32),
        pltpu.VMEM((_CH,), jnp.int32),
        pltpu.VMEM((4 * _L * 32,), jnp.float32),
        pltpu.VMEM((128,), jnp.float32),
        pltpu.VMEM((8 * _BB,), jnp.int32),
        pltpu.VMEM((_BA,), jnp.int32),
        pltpu.VMEM((_BB,), jnp.int32),
        pltpu.SemaphoreType.DMA,
    ],
)
def _stats(loss_hbm, bits_hbm, tgt_hbm, ha_hbm, hb_hbm, out_hbm,
           bi0_v, bi1_v, lo0_v, lo1_v, tg0_v, tg1_v, tbl_v, stg_v, buf_v,
           acca_v, accb_v, sem):
    wid = _wid()
    nch = _PT // _CH
    bi = (bi0_v, bi1_v)
    lo = (lo0_v, lo1_v)
    tg = (tg0_v, tg1_v)

    def start(c, b):
        base = pl.multiple_of(wid * _PT + c * _CH, 8)
        return (pltpu.async_copy(bits_hbm.at[pl.ds(base, _CH)], bi[b], sem),
                pltpu.async_copy(loss_hbm.at[pl.ds(base, _CH)], lo[b], sem),
                pltpu.async_copy(tgt_hbm.at[pl.ds(base, _CH)], tg[b], sem))

    hs = start(0, 0)
    _reduce_rows(ha_hbm, buf_v, acca_v, _BA)
    ba, above_a = _find_kth(acca_v, _BA, _K)
    _reduce_rows(hb_hbm, buf_v, accb_v, _BB)
    bb, _ = _find_kth(accb_v, _BB, _K - above_a)
    pfx = ba * _BB + bb

    _zero(tbl_v, 4 * _L * 32, jnp.float32)
    lane32 = lax.iota(jnp.int32, _L) * 32
    onesf = jnp.ones((_L,), jnp.float32)

    for c in range(nch):
        for h in hs:
            h.wait()
        if c + 1 < nch:
            hs = start(c + 1, (c + 1) % 2)
        bb_v, ll_v, tt_v = bi[c % 2], lo[c % 2], tg[c % 2]

        def body(i, _, bb_v=bb_v, ll_v=ll_v, tt_v=tt_v):
            bits = bb_v[pl.ds(i * _L, _L)]
            v = ll_v[pl.ds(i * _L, _L)]
            t = tt_v[pl.ds(i * _L, _L)]
            hi = bits >> 10
            mg = hi > pfx
            ma = hi >= pfx
            idx = lane32 + t
            off1 = jnp.where(mg, 0, 3 * _L * 32)
            off2 = jnp.where(mg, _L * 32, 2 * _L * 32)
            plsc.addupdate_scatter(tbl_v, [idx + off1], onesf, mask=ma)
            plsc.addupdate_scatter(tbl_v, [idx + off2], v, mask=ma)
            return 0

        lax.fori_loop(0, _CH // _L, body, 0)

    for r, so in ((0, 0), (1, 32), (3, 64), (2, 96)):
        def rbody(j, _, r=r, so=so):
            acc = tbl_v[pl.ds(r * _L * 32 + j * _L, _L)]
            for l in range(1, _L):
                acc = acc + tbl_v[pl.ds(r * _L * 32 + l * 32 + j * _L, _L)]
            stg_v[pl.ds(so + j * _L, _L)] = acc
            return 0

        lax.fori_loop(0, 2, rbody, 0)
    pltpu.sync_copy(stg_v,
                    out_hbm.at[pl.ds(pl.multiple_of(wid * 128, 8), 128)])


def _comb_body(st_ref, out_ref):
    x = st_ref[...]
    cnt = jnp.sum(x[:, 0:32], axis=0)
    s = jnp.sum(x[:, 32:64], axis=0)
    t = jnp.sum(x[:, 64:96], axis=0)
    stie = jnp.sum(x[:, 96:128], axis=0)
    r = _K - jnp.sum(cnt)
    ii = lax.broadcasted_iota(jnp.int32, (32, 32), 0)
    jj = lax.broadcasted_iota(jnp.int32, (32, 32), 1)
    pre = jnp.sum(jnp.where(ii < jj, t[:, None], 0.0), axis=0)
    a = jnp.clip(r - pre, 0.0, t)
    cnt_tot = cnt + a
    s_tot = s + a * (stie / jnp.maximum(t, 1.0))
    contrib = s_tot * lax.rsqrt(cnt_tot + 1e-8)
    out_ref[...] = jnp.reshape(jnp.sum(contrib) * (1.0 / _K ** 0.5), (1, 1))


def _combine(stats):
    return pl.pallas_call(
        _comb_body,
        in_specs=[pl.BlockSpec((_NW, 128), lambda: (0, 0))],
        out_shape=jax.ShapeDtypeStruct((1, 1), jnp.float32),
    )(stats)


def kernel(logits, targets):
    loss_f, loss_i, tflat = _ce_losses(logits, targets)
    ha = _hist_a(loss_i)
    hb = _hist_b(loss_i, ha)
    st = _stats(loss_f, loss_i, tflat, ha, hb)
    return _combine(st.reshape(_NW, 128)).reshape(())


# speedup vs baseline: 54.4039x; 1.3845x over previous
import functools

import jax
import jax.numpy as jnp
from jax import lax
from jax.experimental import pallas as pl
from jax.experimental.pallas import tpu as pltpu
from jax.experimental.pallas import tpu_sc as plsc

_C = 19
_B, _H, _W = 8, 512, 512
_N = _B * _H * _W
_K = int(0.2 * _N)
_NC, _NS, _L = 2, 16, 16
_NW = _NC * _NS
_PT = _N // _NW

_BA = 1024
_BB = 2048
_BC = 1024


_BH = 64


def _ce_body(lg_ref, tg_ref, out_f_ref, out_i_ref, out_t_ref):
    x = lg_ref[0]
    t = tg_ref[0]
    m = jnp.max(x, axis=0)
    s = jnp.sum(jnp.exp(x - m[None]), axis=0)
    cls = lax.broadcasted_iota(jnp.int32, x.shape, 0)
    xt = jnp.sum(jnp.where(cls == t[None], x, 0.0), axis=0)
    loss = jnp.maximum(m + jnp.log(s) - xt, 0.0)
    out_f_ref[...] = loss.reshape(_BH * _W)
    out_i_ref[...] = lax.bitcast_convert_type(loss, jnp.int32).reshape(_BH * _W)
    out_t_ref[...] = t.reshape(_BH * _W)


def _ce_losses(logits, targets):
    flat = pl.BlockSpec((_BH * _W,), lambda b, h: (b * (_H // _BH) + h,))
    return pl.pallas_call(
        _ce_body,
        grid=(_B, _H // _BH),
        in_specs=[
            pl.BlockSpec((1, _C, _BH, _W), lambda b, h: (b, 0, h, 0)),
            pl.BlockSpec((1, _BH, _W), lambda b, h: (b, h, 0)),
        ],
        out_specs=[flat, flat, flat],
        out_shape=[
            jax.ShapeDtypeStruct((_N,), jnp.float32),
            jax.ShapeDtypeStruct((_N,), jnp.int32),
            jax.ShapeDtypeStruct((_N,), jnp.int32),
        ],
    )(logits, targets)


_MESH = plsc.VectorSubcoreMesh(core_axis_name="c", subcore_axis_name="s")


def _wid():
    return lax.axis_index("s") * _NC + lax.axis_index("c")


def _zero(ref, n, dtype):
    z = jnp.zeros((_L,), dtype)

    def body(j, _):
        ref[pl.ds(j * _L, _L)] = z
        return 0

    lax.fori_loop(0, n // _L, body, 0)


def _reduce_rows(hist_hbm, buf_v, acc_v, nbins):
    rows = 8
    _zero(acc_v, nbins, jnp.int32)

    def chunk(ci, _):
        src = hist_hbm.at[pl.ds(pl.multiple_of(ci * (rows * nbins), 8),
                                rows * nbins)]
        pltpu.sync_copy(src, buf_v.at[pl.ds(0, rows * nbins)])

        def jbody(j, __):
            acc = acc_v[pl.ds(j * _L, _L)]
            for rr in range(rows):
                acc = acc + buf_v[pl.ds(rr * nbins + j * _L, _L)]
            acc_v[pl.ds(j * _L, _L)] = acc
            return 0

        lax.fori_loop(0, nbins // _L, jbody, 0)
        return 0

    lax.fori_loop(0, _NW // rows, chunk, 0)


def _find_kth(acc_v, nbins, kk):
    nch = nbins // _L
    lane = lax.iota(jnp.int32, _L)

    def body(i, carry):
        found, bfound, above, cum = carry
        j = nch - 1 - i
        v = acc_v[pl.ds(j * _L, _L)]
        rv = lax.rev(v, (0,))
        cs = plsc.cumsum(rv)
        tot = jnp.max(cs)
        hit = (cum + cs) >= kk
        anyhit = jnp.max(hit.astype(jnp.int32)) > 0
        ps = jnp.max(plsc.all_reduce_ffs(hit))
        bin_here = j * _L + (_L - 1) - ps
        above_here = cum + jnp.sum(jnp.where(lane < ps, rv, 0))
        take = jnp.logical_and(anyhit, found == 0)
        return (jnp.where(take, 1, found),
                jnp.where(take, bin_here, bfound),
                jnp.where(take, above_here, above),
                cum + tot)

    _, b, above, _ = lax.fori_loop(0, nch, body, (0, 0, 0, 0))
    return b, above


def _lane_reduce_store(tbl_v, red_v, nbins):
    def rbody(j, _):
        acc = tbl_v[pl.ds(j * _L, _L)]
        for l in range(1, _L):
            acc = acc + tbl_v[pl.ds(l * nbins + j * _L, _L)]
        red_v[pl.ds(j * _L, _L)] = acc
        return 0

    lax.fori_loop(0, nbins // _L, rbody, 0)


@functools.partial(
    pl.kernel,
    out_type=jax.ShapeDtypeStruct((_NW * _BA,), jnp.int32),
    mesh=_MESH,
    compiler_params=pltpu.CompilerParams(needs_layout_passes=False),
    scratch_types=[
        pltpu.VMEM((_PT,), jnp.int32),
        pltpu.VMEM((_L * _BA,), jnp.int32),
        pltpu.VMEM((_BA,), jnp.int32),
    ],
)
def _hist_a(bits_hbm, out_hbm, data_v, tbl_v, red_v):
    wid = _wid()
    pltpu.sync_copy(bits_hbm.at[pl.ds(pl.multiple_of(wid * _PT, 8), _PT)],
                    data_v)
    _zero(tbl_v, _L * _BA, jnp.int32)
    lb = lax.iota(jnp.int32, _L) * _BA
    ones = jnp.ones((_L,), jnp.int32)

    @plsc.parallel_loop(0, _PT // _L, 1, unroll=8)
    def body(i):
        bits = data_v[pl.ds(i * _L, _L)]
        plsc.addupdate_scatter(tbl_v, [lb + (bits >> 21)], ones)

    _lane_reduce_store(tbl_v, red_v, _BA)
    pltpu.sync_copy(red_v,
                    out_hbm.at[pl.ds(pl.multiple_of(wid * _BA, 8), _BA)])


@functools.partial(
    pl.kernel,
    out_type=jax.ShapeDtypeStruct((_NW * _BB,), jnp.int32),
    mesh=_MESH,
    compiler_params=pltpu.CompilerParams(needs_layout_passes=False),
    scratch_types=[
        pltpu.VMEM((_PT,), jnp.int32),
        pltpu.VMEM((_L * _BB,), jnp.int32),
        pltpu.VMEM((_BB,), jnp.int32),
        pltpu.VMEM((8 * _BA,), jnp.int32),
        pltpu.VMEM((_BA,), jnp.int32),
    ],
)
def _hist_b(bits_hbm, ha_hbm, out_hbm, data_v, tbl_v, red_v, buf_v, acca_v):
    wid = _wid()
    _reduce_rows(ha_hbm, buf_v, acca_v, _BA)
    ba, _ = _find_kth(acca_v, _BA, _K)
    pltpu.sync_copy(bits_hbm.at[pl.ds(pl.multiple_of(wid * _PT, 8), _PT)],
                    data_v)
    _zero(tbl_v, _L * _BB, jnp.int32)
    lb = lax.iota(jnp.int32, _L) * _BB
    ones = jnp.ones((_L,), jnp.int32)

    @plsc.parallel_loop(0, _PT // _L, 1, unroll=8)
    def body(i):
        bits = data_v[pl.ds(i * _L, _L)]
        m = (bits >> 21) == ba
        plsc.addupdate_scatter(tbl_v, [lb + ((bits >> 10) & 0x7FF)], ones,
                               mask=m)

    _lane_reduce_store(tbl_v, red_v, _BB)
    pltpu.sync_copy(red_v,
                    out_hbm.at[pl.ds(pl.multiple_of(wid * _BB, 8), _BB)])


_CH = 16384


@functools.partial(
    pl.kernel,
    out_type=jax.ShapeDtypeStruct((_NW * 128,), jnp.float32),
    mesh=_MESH,
    compiler_params=pltpu.CompilerParams(needs_layout_passes=False),
    scratch_types=[
        pltpu.VMEM((_CH,), jnp.int32),
        pltpu.VMEM((_CH,), jnp.int32),
        pltpu.VMEM((_CH,), jnp.float32),
        pltpu.VMEM((_CH,), jnp.float32),
        pltpu.VMEM((_CH,), jnp.int32),
        pltpu.VMEM((_CH,), jnp.int32),
        pltpu.VMEM((4 * _L * 32,), jnp.float32),
        pltpu.VMEM((128,), jnp.float32),
        pltpu.VMEM((8 * _BB,), jnp.int32),
        pltpu.VMEM((_BA,), jnp.int32),
        pltpu.VMEM((_BB,), jnp.int32),
        pltpu.SemaphoreType.DMA,
    ],
)
def _stats(loss_hbm, bits_hbm, tgt_hbm, ha_hbm, hb_hbm, out_hbm,
           bi0_v, bi1_v, lo0_v, lo1_v, tg0_v, tg1_v, tbl_v, stg_v, buf_v,
           acca_v, accb_v, sem):
    wid = _wid()
    nch = _PT // _CH
    bi = (bi0_v, bi1_v)
    lo = (lo0_v, lo1_v)
    tg = (tg0_v, tg1_v)

    def start(c, b):
        base = pl.multiple_of(wid * _PT + c * _CH, 8)
        return (pltpu.async_copy(bits_hbm.at[pl.ds(base, _CH)], bi[b], sem),
                pltpu.async_copy(loss_hbm.at[pl.ds(base, _CH)], lo[b], sem),
                pltpu.async_copy(tgt_hbm.at[pl.ds(base, _CH)], tg[b], sem))

    hs = start(0, 0)
    _reduce_rows(ha_hbm, buf_v, acca_v, _BA)
    ba, above_a = _find_kth(acca_v, _BA, _K)
    _reduce_rows(hb_hbm, buf_v, accb_v, _BB)
    bb, _ = _find_kth(accb_v, _BB, _K - above_a)
    pfx = ba * _BB + bb

    _zero(tbl_v, 4 * _L * 32, jnp.float32)
    lane32 = lax.iota(jnp.int32, _L) * 32
    onesf = jnp.ones((_L,), jnp.float32)

    for c in range(nch):
        for h in hs:
            h.wait()
        if c + 1 < nch:
            hs = start(c + 1, (c + 1) % 2)
        bb_v, ll_v, tt_v = bi[c % 2], lo[c % 2], tg[c % 2]

        @plsc.parallel_loop(0, _CH // _L, 1, unroll=8)
        def body(i, bb_v=bb_v, ll_v=ll_v, tt_v=tt_v):
            bits = bb_v[pl.ds(i * _L, _L)]
            v = ll_v[pl.ds(i * _L, _L)]
            t = tt_v[pl.ds(i * _L, _L)]
            hi = bits >> 10
            mg = hi > pfx
            ma = hi >= pfx
            idx = lane32 + t
            off1 = jnp.where(mg, 0, 3 * _L * 32)
            off2 = jnp.where(mg, _L * 32, 2 * _L * 32)
            plsc.addupdate_scatter(tbl_v, [idx + off1], onesf, mask=ma)
            plsc.addupdate_scatter(tbl_v, [idx + off2], v, mask=ma)

    for r, so in ((0, 0), (1, 32), (3, 64), (2, 96)):
        def rbody(j, _, r=r, so=so):
            acc = tbl_v[pl.ds(r * _L * 32 + j * _L, _L)]
            for l in range(1, _L):
                acc = acc + tbl_v[pl.ds(r * _L * 32 + l * 32 + j * _L, _L)]
            stg_v[pl.ds(so + j * _L, _L)] = acc
            return 0

        lax.fori_loop(0, 2, rbody, 0)
    pltpu.sync_copy(stg_v,
                    out_hbm.at[pl.ds(pl.multiple_of(wid * 128, 8), 128)])


def _comb_body(st_ref, out_ref):
    x = st_ref[...]
    cnt = jnp.sum(x[:, 0:32], axis=0)
    s = jnp.sum(x[:, 32:64], axis=0)
    t = jnp.sum(x[:, 64:96], axis=0)
    stie = jnp.sum(x[:, 96:128], axis=0)
    r = _K - jnp.sum(cnt)
    ii = lax.broadcasted_iota(jnp.int32, (32, 32), 0)
    jj = lax.broadcasted_iota(jnp.int32, (32, 32), 1)
    pre = jnp.sum(jnp.where(ii < jj, t[:, None], 0.0), axis=0)
    a = jnp.clip(r - pre, 0.0, t)
    cnt_tot = cnt + a
    s_tot = s + a * (stie / jnp.maximum(t, 1.0))
    contrib = s_tot * lax.rsqrt(cnt_tot + 1e-8)
    out_ref[...] = jnp.reshape(jnp.sum(contrib) * (1.0 / _K ** 0.5), (1, 1))


def _combine(stats):
    return pl.pallas_call(
        _comb_body,
        in_specs=[pl.BlockSpec((_NW, 128), lambda: (0, 0))],
        out_shape=jax.ShapeDtypeStruct((1, 1), jnp.float32),
    )(stats)


def kernel(logits, targets):
    loss_f, loss_i, tflat = _ce_losses(logits, targets)
    ha = _hist_a(loss_i)
    hb = _hist_b(loss_i, ha)
    st = _stats(loss_f, loss_i, tflat, ha, hb)
    return _combine(st.reshape(_NW, 128)).reshape(())


# speedup vs baseline: 58.9171x; 1.0830x over previous
import functools

import jax
import jax.numpy as jnp
from jax import lax
from jax.experimental import pallas as pl
from jax.experimental.pallas import tpu as pltpu
from jax.experimental.pallas import tpu_sc as plsc

_C = 19
_B, _H, _W = 8, 512, 512
_N = _B * _H * _W
_K = int(0.2 * _N)
_NC, _NS, _L = 2, 16, 16
_NW = _NC * _NS
_PT = _N // _NW

_BA = 1024
_BB = 2048
_BC = 1024


_BH = 64


def _ce_body(lg_ref, tg_ref, out_f_ref, out_i_ref, out_t_ref):
    x = lg_ref[0]
    t = tg_ref[0]
    m = jnp.max(x, axis=0)
    s = jnp.sum(jnp.exp(x - m[None]), axis=0)
    cls = lax.broadcasted_iota(jnp.int32, x.shape, 0)
    xt = jnp.sum(jnp.where(cls == t[None], x, 0.0), axis=0)
    loss = jnp.maximum(m + jnp.log(s) - xt, 0.0)
    out_f_ref[...] = loss.reshape(_BH * _W)
    out_i_ref[...] = lax.bitcast_convert_type(loss, jnp.int32).reshape(_BH * _W)
    out_t_ref[...] = t.reshape(_BH * _W)


def _ce_losses(logits, targets):
    flat = pl.BlockSpec((_BH * _W,), lambda b, h: (b * (_H // _BH) + h,))
    return pl.pallas_call(
        _ce_body,
        grid=(_B, _H // _BH),
        in_specs=[
            pl.BlockSpec((1, _C, _BH, _W), lambda b, h: (b, 0, h, 0)),
            pl.BlockSpec((1, _BH, _W), lambda b, h: (b, h, 0)),
        ],
        out_specs=[flat, flat, flat],
        out_shape=[
            jax.ShapeDtypeStruct((_N,), jnp.float32),
            jax.ShapeDtypeStruct((_N,), jnp.int32),
            jax.ShapeDtypeStruct((_N,), jnp.int32),
        ],
    )(logits, targets)


_MESH = plsc.VectorSubcoreMesh(core_axis_name="c", subcore_axis_name="s")


def _wid():
    return lax.axis_index("s") * _NC + lax.axis_index("c")


def _zero(ref, n, dtype):
    z = jnp.zeros((_L,), dtype)

    @plsc.parallel_loop(0, n // _L, 1, unroll=8)
    def body(j):
        ref[pl.ds(j * _L, _L)] = z


def _reduce_rows(hist_hbm, buf_v, acc_v, nbins):
    rows = 8
    _zero(acc_v, nbins, jnp.int32)

    def chunk(ci, _):
        src = hist_hbm.at[pl.ds(pl.multiple_of(ci * (rows * nbins), 8),
                                rows * nbins)]
        pltpu.sync_copy(src, buf_v.at[pl.ds(0, rows * nbins)])

        @plsc.parallel_loop(0, nbins // _L, 1, unroll=4)
        def jbody(j):
            acc = acc_v[pl.ds(j * _L, _L)]
            for rr in range(rows):
                acc = acc + buf_v[pl.ds(rr * nbins + j * _L, _L)]
            acc_v[pl.ds(j * _L, _L)] = acc
        return 0

    lax.fori_loop(0, _NW // rows, chunk, 0)


def _find_kth(acc_v, nbins, kk):
    nch = nbins // _L
    lane = lax.iota(jnp.int32, _L)

    def body(i, carry):
        found, bfound, above, cum = carry
        j = nch - 1 - i
        v = acc_v[pl.ds(j * _L, _L)]
        rv = lax.rev(v, (0,))
        cs = plsc.cumsum(rv)
        tot = jnp.max(cs)
        hit = (cum + cs) >= kk
        anyhit = jnp.max(hit.astype(jnp.int32)) > 0
        ps = jnp.max(plsc.all_reduce_ffs(hit))
        bin_here = j * _L + (_L - 1) - ps
        above_here = cum + jnp.sum(jnp.where(lane < ps, rv, 0))
        take = jnp.logical_and(anyhit, found == 0)
        return (jnp.where(take, 1, found),
                jnp.where(take, bin_here, bfound),
                jnp.where(take, above_here, above),
                cum + tot)

    _, b, above, _ = lax.fori_loop(0, nch, body, (0, 0, 0, 0))
    return b, above


def _lane_reduce_store(tbl_v, red_v, nbins):
    @plsc.parallel_loop(0, nbins // _L, 1, unroll=4)
    def rbody(j):
        acc = tbl_v[pl.ds(j * _L, _L)]
        for l in range(1, _L):
            acc = acc + tbl_v[pl.ds(l * nbins + j * _L, _L)]
        red_v[pl.ds(j * _L, _L)] = acc


@functools.partial(
    pl.kernel,
    out_type=jax.ShapeDtypeStruct((_NW * _BA,), jnp.int32),
    mesh=_MESH,
    compiler_params=pltpu.CompilerParams(needs_layout_passes=False),
    scratch_types=[
        pltpu.VMEM((_PT,), jnp.int32),
        pltpu.VMEM((_L * _BA,), jnp.int32),
        pltpu.VMEM((_BA,), jnp.int32),
    ],
)
def _hist_a(bits_hbm, out_hbm, data_v, tbl_v, red_v):
    wid = _wid()
    pltpu.sync_copy(bits_hbm.at[pl.ds(pl.multiple_of(wid * _PT, 8), _PT)],
                    data_v)
    _zero(tbl_v, _L * _BA, jnp.int32)
    lb = lax.iota(jnp.int32, _L) * _BA
    ones = jnp.ones((_L,), jnp.int32)

    @plsc.parallel_loop(0, _PT // _L, 1, unroll=8)
    def body(i):
        bits = data_v[pl.ds(i * _L, _L)]
        plsc.addupdate_scatter(tbl_v, [lb + (bits >> 21)], ones)

    _lane_reduce_store(tbl_v, red_v, _BA)
    pltpu.sync_copy(red_v,
                    out_hbm.at[pl.ds(pl.multiple_of(wid * _BA, 8), _BA)])


@functools.partial(
    pl.kernel,
    out_type=jax.ShapeDtypeStruct((_NW * _BB,), jnp.int32),
    mesh=_MESH,
    compiler_params=pltpu.CompilerParams(needs_layout_passes=False),
    scratch_types=[
        pltpu.VMEM((_PT,), jnp.int32),
        pltpu.VMEM((_L * _BB,), jnp.int32),
        pltpu.VMEM((_BB,), jnp.int32),
        pltpu.VMEM((8 * _BA,), jnp.int32),
        pltpu.VMEM((_BA,), jnp.int32),
        pltpu.SemaphoreType.DMA,
    ],
)
def _hist_b(bits_hbm, ha_hbm, out_hbm, data_v, tbl_v, red_v, buf_v, acca_v,
            dsem):
    wid = _wid()
    cp = pltpu.async_copy(
        bits_hbm.at[pl.ds(pl.multiple_of(wid * _PT, 8), _PT)], data_v, dsem)
    _reduce_rows(ha_hbm, buf_v, acca_v, _BA)
    ba, _ = _find_kth(acca_v, _BA, _K)
    _zero(tbl_v, _L * _BB, jnp.int32)
    cp.wait()
    lb = lax.iota(jnp.int32, _L) * _BB
    ones = jnp.ones((_L,), jnp.int32)

    @plsc.parallel_loop(0, _PT // _L, 1, unroll=8)
    def body(i):
        bits = data_v[pl.ds(i * _L, _L)]
        m = (bits >> 21) == ba
        plsc.addupdate_scatter(tbl_v, [lb + ((bits >> 10) & 0x7FF)], ones,
                               mask=m)

    _lane_reduce_store(tbl_v, red_v, _BB)
    pltpu.sync_copy(red_v,
                    out_hbm.at[pl.ds(pl.multiple_of(wid * _BB, 8), _BB)])


_CH = 16384


@functools.partial(
    pl.kernel,
    out_type=jax.ShapeDtypeStruct((_NW * 128,), jnp.float32),
    mesh=_MESH,
    compiler_params=pltpu.CompilerParams(needs_layout_passes=False),
    scratch_types=[
        pltpu.VMEM((_CH,), jnp.int32),
        pltpu.VMEM((_CH,), jnp.int32),
        pltpu.VMEM((_CH,), jnp.float32),
        pltpu.VMEM((_CH,), jnp.float32),
        pltpu.VMEM((_CH,), jnp.int32),
        pltpu.VMEM((_CH,), jnp.int32),
        pltpu.VMEM((4 * _L * 32,), jnp.float32),
        pltpu.VMEM((128,), jnp.float32),
        pltpu.VMEM((8 * _BB,), jnp.int32),
        pltpu.VMEM((_BA,), jnp.int32),
        pltpu.VMEM((_BB,), jnp.int32),
        pltpu.SemaphoreType.DMA,
    ],
)
def _stats(loss_hbm, bits_hbm, tgt_hbm, ha_hbm, hb_hbm, out_hbm,
           bi0_v, bi1_v, lo0_v, lo1_v, tg0_v, tg1_v, tbl_v, stg_v, buf_v,
           acca_v, accb_v, sem):
    wid = _wid()
    nch = _PT // _CH
    bi = (bi0_v, bi1_v)
    lo = (lo0_v, lo1_v)
    tg = (tg0_v, tg1_v)

    def start(c, b):
        base = pl.multiple_of(wid * _PT + c * _CH, 8)
        return (pltpu.async_copy(bits_hbm.at[pl.ds(base, _CH)], bi[b], sem),
                pltpu.async_copy(loss_hbm.at[pl.ds(base, _CH)], lo[b], sem),
                pltpu.async_copy(tgt_hbm.at[pl.ds(base, _CH)], tg[b], sem))

    hs = start(0, 0)
    _reduce_rows(ha_hbm, buf_v, acca_v, _BA)
    ba, above_a = _find_kth(acca_v, _BA, _K)
    _reduce_rows(hb_hbm, buf_v, accb_v, _BB)
    bb, _ = _find_kth(accb_v, _BB, _K - above_a)
    pfx = ba * _BB + bb

    _zero(tbl_v, 4 * _L * 32, jnp.float32)
    lane32 = lax.iota(jnp.int32, _L) * 32
    onesf = jnp.ones((_L,), jnp.float32)

    for c in range(nch):
        for h in hs:
            h.wait()
        if c + 1 < nch:
            hs = start(c + 1, (c + 1) % 2)
        bb_v, ll_v, tt_v = bi[c % 2], lo[c % 2], tg[c % 2]

        @plsc.parallel_loop(0, _CH // _L, 1, unroll=8)
        def body(i, bb_v=bb_v, ll_v=ll_v, tt_v=tt_v):
            bits = bb_v[pl.ds(i * _L, _L)]
            v = ll_v[pl.ds(i * _L, _L)]
            t = tt_v[pl.ds(i * _L, _L)]
            hi = bits >> 10
            mg = hi > pfx
            ma = hi >= pfx
            idx = lane32 + t
            off1 = jnp.where(mg, 0, 3 * _L * 32)
            off2 = jnp.where(mg, _L * 32, 2 * _L * 32)
            plsc.addupdate_scatter(tbl_v, [idx + off1], onesf, mask=ma)
            plsc.addupdate_scatter(tbl_v, [idx + off2], v, mask=ma)

    for r, so in ((0, 0), (1, 32), (3, 64), (2, 96)):
        def rbody(j, _, r=r, so=so):
            acc = tbl_v[pl.ds(r * _L * 32 + j * _L, _L)]
            for l in range(1, _L):
                acc = acc + tbl_v[pl.ds(r * _L * 32 + l * 32 + j * _L, _L)]
            stg_v[pl.ds(so + j * _L, _L)] = acc
            return 0

        lax.fori_loop(0, 2, rbody, 0)
    pltpu.sync_copy(stg_v,
                    out_hbm.at[pl.ds(pl.multiple_of(wid * 128, 8), 128)])


def _comb_body(st_ref, out_ref):
    x = st_ref[...]
    cnt = jnp.sum(x[:, 0:32], axis=0)
    s = jnp.sum(x[:, 32:64], axis=0)
    t = jnp.sum(x[:, 64:96], axis=0)
    stie = jnp.sum(x[:, 96:128], axis=0)
    r = _K - jnp.sum(cnt)
    ii = lax.broadcasted_iota(jnp.int32, (32, 32), 0)
    jj = lax.broadcasted_iota(jnp.int32, (32, 32), 1)
    pre = jnp.sum(jnp.where(ii < jj, t[:, None], 0.0), axis=0)
    a = jnp.clip(r - pre, 0.0, t)
    cnt_tot = cnt + a
    s_tot = s + a * (stie / jnp.maximum(t, 1.0))
    contrib = s_tot * lax.rsqrt(cnt_tot + 1e-8)
    out_ref[...] = jnp.reshape(jnp.sum(contrib) * (1.0 / _K ** 0.5), (1, 1))


def _combine(stats):
    return pl.pallas_call(
        _comb_body,
        in_specs=[pl.BlockSpec((_NW, 128), lambda: (0, 0))],
        out_shape=jax.ShapeDtypeStruct((1, 1), jnp.float32),
    )(stats)


def kernel(logits, targets):
    loss_f, loss_i, tflat = _ce_losses(logits, targets)
    ha = _hist_a(loss_i)
    hb = _hist_b(loss_i, ha)
    st = _stats(loss_f, loss_i, tflat, ha, hb)
    return _combine(st.reshape(_NW, 128)).reshape(())


# speedup vs baseline: 64.9128x; 1.1018x over previous
import functools

import jax
import jax.numpy as jnp
from jax import lax
from jax.experimental import pallas as pl
from jax.experimental.pallas import tpu as pltpu
from jax.experimental.pallas import tpu_sc as plsc

_C = 19
_B, _H, _W = 8, 512, 512
_N = _B * _H * _W
_K = int(0.2 * _N)
_NC, _NS, _L = 2, 16, 16
_NW = _NC * _NS
_PT = _N // _NW

_BA = 1024
_BB = 2048
_BC = 1024


_BH = 128


def _ce_body(lg_ref, tg_ref, out_f_ref, out_i_ref, out_t_ref):
    x = lg_ref[0]
    t = tg_ref[0]
    m = jnp.max(x, axis=0)
    s = jnp.sum(jnp.exp(x - m[None]), axis=0)
    cls = lax.broadcasted_iota(jnp.int32, x.shape, 0)
    xt = jnp.sum(jnp.where(cls == t[None], x, 0.0), axis=0)
    loss = jnp.maximum(m + jnp.log(s) - xt, 0.0)
    out_f_ref[...] = loss.reshape(_BH * _W)
    out_i_ref[...] = lax.bitcast_convert_type(loss, jnp.int32).reshape(_BH * _W)
    out_t_ref[...] = t.reshape(_BH * _W)


def _ce_losses(logits, targets):
    flat = pl.BlockSpec((_BH * _W,), lambda b, h: (b * (_H // _BH) + h,))
    return pl.pallas_call(
        _ce_body,
        grid=(_B, _H // _BH),
        in_specs=[
            pl.BlockSpec((1, _C, _BH, _W), lambda b, h: (b, 0, h, 0)),
            pl.BlockSpec((1, _BH, _W), lambda b, h: (b, h, 0)),
        ],
        out_specs=[flat, flat, flat],
        out_shape=[
            jax.ShapeDtypeStruct((_N,), jnp.float32),
            jax.ShapeDtypeStruct((_N,), jnp.int32),
            jax.ShapeDtypeStruct((_N,), jnp.int32),
        ],
    )(logits, targets)


_MESH = plsc.VectorSubcoreMesh(core_axis_name="c", subcore_axis_name="s")


def _wid():
    return lax.axis_index("s") * _NC + lax.axis_index("c")


def _zero(ref, n, dtype):
    z = jnp.zeros((_L,), dtype)

    @plsc.parallel_loop(0, n // _L, 1, unroll=8)
    def body(j):
        ref[pl.ds(j * _L, _L)] = z


def _reduce_rows(hist_hbm, buf_v, acc_v, nbins):
    rows = 8
    _zero(acc_v, nbins, jnp.int32)

    def chunk(ci, _):
        src = hist_hbm.at[pl.ds(pl.multiple_of(ci * (rows * nbins), 8),
                                rows * nbins)]
        pltpu.sync_copy(src, buf_v.at[pl.ds(0, rows * nbins)])

        @plsc.parallel_loop(0, nbins // _L, 1, unroll=4)
        def jbody(j):
            acc = acc_v[pl.ds(j * _L, _L)]
            for rr in range(rows):
                acc = acc + buf_v[pl.ds(rr * nbins + j * _L, _L)]
            acc_v[pl.ds(j * _L, _L)] = acc
        return 0

    lax.fori_loop(0, _NW // rows, chunk, 0)


def _find_kth(acc_v, nbins, kk):
    nch = nbins // _L
    lane = lax.iota(jnp.int32, _L)

    def body(i, carry):
        found, bfound, above, cum = carry
        j = nch - 1 - i
        v = acc_v[pl.ds(j * _L, _L)]
        rv = lax.rev(v, (0,))
        cs = plsc.cumsum(rv)
        tot = jnp.max(cs)
        hit = (cum + cs) >= kk
        anyhit = jnp.max(hit.astype(jnp.int32)) > 0
        ps = jnp.max(plsc.all_reduce_ffs(hit))
        bin_here = j * _L + (_L - 1) - ps
        above_here = cum + jnp.sum(jnp.where(lane < ps, rv, 0))
        take = jnp.logical_and(anyhit, found == 0)
        return (jnp.where(take, 1, found),
                jnp.where(take, bin_here, bfound),
                jnp.where(take, above_here, above),
                cum + tot)

    _, b, above, _ = lax.fori_loop(0, nch, body, (0, 0, 0, 0))
    return b, above


def _lane_reduce_store(tbl_v, red_v, nbins):
    @plsc.parallel_loop(0, nbins // _L, 1, unroll=4)
    def rbody(j):
        acc = tbl_v[pl.ds(j * _L, _L)]
        for l in range(1, _L):
            acc = acc + tbl_v[pl.ds(l * nbins + j * _L, _L)]
        red_v[pl.ds(j * _L, _L)] = acc


@functools.partial(
    pl.kernel,
    out_type=jax.ShapeDtypeStruct((_NW * _BA,), jnp.int32),
    mesh=_MESH,
    compiler_params=pltpu.CompilerParams(needs_layout_passes=False),
    scratch_types=[
        pltpu.VMEM((_PT,), jnp.int32),
        pltpu.VMEM((_L * _BA,), jnp.int32),
        pltpu.VMEM((_BA,), jnp.int32),
        pltpu.SemaphoreType.DMA,
    ],
)
def _hist_a(bits_hbm, out_hbm, data_v, tbl_v, red_v, dsem):
    wid = _wid()
    cp = pltpu.async_copy(
        bits_hbm.at[pl.ds(pl.multiple_of(wid * _PT, 8), _PT)], data_v, dsem)
    _zero(tbl_v, _L * _BA, jnp.int32)
    cp.wait()
    lb = lax.iota(jnp.int32, _L) * _BA
    ones = jnp.ones((_L,), jnp.int32)

    @plsc.parallel_loop(0, _PT // _L, 1, unroll=8)
    def body(i):
        bits = data_v[pl.ds(i * _L, _L)]
        plsc.addupdate_scatter(tbl_v, [lb + (bits >> 21)], ones)

    _lane_reduce_store(tbl_v, red_v, _BA)
    pltpu.sync_copy(red_v,
                    out_hbm.at[pl.ds(pl.multiple_of(wid * _BA, 8), _BA)])


@functools.partial(
    pl.kernel,
    out_type=jax.ShapeDtypeStruct((_NW * _BB,), jnp.int32),
    mesh=_MESH,
    compiler_params=pltpu.CompilerParams(needs_layout_passes=False),
    scratch_types=[
        pltpu.VMEM((_PT,), jnp.int32),
        pltpu.VMEM((_L * _BB,), jnp.int32),
        pltpu.VMEM((_BB,), jnp.int32),
        pltpu.VMEM((8 * _BA,), jnp.int32),
        pltpu.VMEM((_BA,), jnp.int32),
        pltpu.SemaphoreType.DMA,
    ],
)
def _hist_b(bits_hbm, ha_hbm, out_hbm, data_v, tbl_v, red_v, buf_v, acca_v,
            dsem):
    wid = _wid()
    cp = pltpu.async_copy(
        bits_hbm.at[pl.ds(pl.multiple_of(wid * _PT, 8), _PT)], data_v, dsem)
    _reduce_rows(ha_hbm, buf_v, acca_v, _BA)
    ba, _ = _find_kth(acca_v, _BA, _K)
    _zero(tbl_v, _L * _BB, jnp.int32)
    cp.wait()
    lb = lax.iota(jnp.int32, _L) * _BB
    ones = jnp.ones((_L,), jnp.int32)

    @plsc.parallel_loop(0, _PT // _L, 1, unroll=8)
    def body(i):
        bits = data_v[pl.ds(i * _L, _L)]
        m = (bits >> 21) == ba
        plsc.addupdate_scatter(tbl_v, [lb + ((bits >> 10) & 0x7FF)], ones,
                               mask=m)

    _lane_reduce_store(tbl_v, red_v, _BB)
    pltpu.sync_copy(red_v,
                    out_hbm.at[pl.ds(pl.multiple_of(wid * _BB, 8), _BB)])


_CH = 16384


@functools.partial(
    pl.kernel,
    out_type=jax.ShapeDtypeStruct((_NW * 128,), jnp.float32),
    mesh=_MESH,
    compiler_params=pltpu.CompilerParams(needs_layout_passes=False),
    scratch_types=[
        pltpu.VMEM((_CH,), jnp.int32),
        pltpu.VMEM((_CH,), jnp.int32),
        pltpu.VMEM((_CH,), jnp.float32),
        pltpu.VMEM((_CH,), jnp.float32),
        pltpu.VMEM((_CH,), jnp.int32),
        pltpu.VMEM((_CH,), jnp.int32),
        pltpu.VMEM((4 * _L * 32,), jnp.float32),
        pltpu.VMEM((128,), jnp.float32),
        pltpu.VMEM((8 * _BB,), jnp.int32),
        pltpu.VMEM((_BA,), jnp.int32),
        pltpu.VMEM((_BB,), jnp.int32),
        pltpu.SemaphoreType.DMA,
    ],
)
def _stats(loss_hbm, bits_hbm, tgt_hbm, ha_hbm, hb_hbm, out_hbm,
           bi0_v, bi1_v, lo0_v, lo1_v, tg0_v, tg1_v, tbl_v, stg_v, buf_v,
           acca_v, accb_v, sem):
    wid = _wid()
    nch = _PT // _CH
    bi = (bi0_v, bi1_v)
    lo = (lo0_v, lo1_v)
    tg = (tg0_v, tg1_v)

    def start(c, b):
        base = pl.multiple_of(wid * _PT + c * _CH, 8)
        return (pltpu.async_copy(bits_hbm.at[pl.ds(base, _CH)], bi[b], sem),
                pltpu.async_copy(loss_hbm.at[pl.ds(base, _CH)], lo[b], sem),
                pltpu.async_copy(tgt_hbm.at[pl.ds(base, _CH)], tg[b], sem))

    hs = start(0, 0)
    _reduce_rows(ha_hbm, buf_v, acca_v, _BA)
    ba, above_a = _find_kth(acca_v, _BA, _K)
    _reduce_rows(hb_hbm, buf_v, accb_v, _BB)
    bb, _ = _find_kth(accb_v, _BB, _K - above_a)
    pfx = ba * _BB + bb

    _zero(tbl_v, 4 * _L * 32, jnp.float32)
    lane32 = lax.iota(jnp.int32, _L) * 32
    onesf = jnp.ones((_L,), jnp.float32)

    for c in range(nch):
        for h in hs:
            h.wait()
        if c + 1 < nch:
            hs = start(c + 1, (c + 1) % 2)
        bb_v, ll_v, tt_v = bi[c % 2], lo[c % 2], tg[c % 2]

        @plsc.parallel_loop(0, _CH // _L, 1, unroll=8)
        def body(i, bb_v=bb_v, ll_v=ll_v, tt_v=tt_v):
            bits = bb_v[pl.ds(i * _L, _L)]
            v = ll_v[pl.ds(i * _L, _L)]
            t = tt_v[pl.ds(i * _L, _L)]
            hi = bits >> 10
            mg = hi > pfx
            ma = hi >= pfx
            idx = lane32 + t
            off1 = jnp.where(mg, 0, 3 * _L * 32)
            off2 = jnp.where(mg, _L * 32, 2 * _L * 32)
            plsc.addupdate_scatter(tbl_v, [idx + off1], onesf, mask=ma)
            plsc.addupdate_scatter(tbl_v, [idx + off2], v, mask=ma)

    for r, so in ((0, 0), (1, 32), (3, 64), (2, 96)):
        def rbody(j, _, r=r, so=so):
            acc = tbl_v[pl.ds(r * _L * 32 + j * _L, _L)]
            for l in range(1, _L):
                acc = acc + tbl_v[pl.ds(r * _L * 32 + l * 32 + j * _L, _L)]
            stg_v[pl.ds(so + j * _L, _L)] = acc
            return 0

        lax.fori_loop(0, 2, rbody, 0)
    pltpu.sync_copy(stg_v,
                    out_hbm.at[pl.ds(pl.multiple_of(wid * 128, 8), 128)])


def _comb_body(st_ref, out_ref):
    x = st_ref[...]
    cnt = jnp.sum(x[:, 0:32], axis=0)
    s = jnp.sum(x[:, 32:64], axis=0)
    t = jnp.sum(x[:, 64:96], axis=0)
    stie = jnp.sum(x[:, 96:128], axis=0)
    r = _K - jnp.sum(cnt)
    ii = lax.broadcasted_iota(jnp.int32, (32, 32), 0)
    jj = lax.broadcasted_iota(jnp.int32, (32, 32), 1)
    pre = jnp.sum(jnp.where(ii < jj, t[:, None], 0.0), axis=0)
    a = jnp.clip(r - pre, 0.0, t)
    cnt_tot = cnt + a
    s_tot = s + a * (stie / jnp.maximum(t, 1.0))
    contrib = s_tot * lax.rsqrt(cnt_tot + 1e-8)
    out_ref[...] = jnp.reshape(jnp.sum(contrib) * (1.0 / _K ** 0.5), (1, 1))


def _combine(stats):
    return pl.pallas_call(
        _comb_body,
        in_specs=[pl.BlockSpec((_NW, 128), lambda: (0, 0))],
        out_shape=jax.ShapeDtypeStruct((1, 1), jnp.float32),
    )(stats)


def kernel(logits, targets):
    loss_f, loss_i, tflat = _ce_losses(logits, targets)
    ha = _hist_a(loss_i)
    hb = _hist_b(loss_i, ha)
    st = _stats(loss_f, loss_i, tflat, ha, hb)
    return _combine(st.reshape(_NW, 128)).reshape(())


# speedup vs baseline: 67.1058x; 1.0338x over previous
import functools

import jax
import jax.numpy as jnp
from jax import lax
from jax.experimental import pallas as pl
from jax.experimental.pallas import tpu as pltpu
from jax.experimental.pallas import tpu_sc as plsc

_C = 19
_B, _H, _W = 8, 512, 512
_N = _B * _H * _W
_K = int(0.2 * _N)
_NC, _NS, _L = 2, 16, 16
_NW = _NC * _NS
_PT = _N // _NW

_BA = 1024
_BB = 2048
_BC = 1024


_BH = 128


def _ce_body(lg_ref, tg_ref, out_f_ref, out_i_ref):
    x = lg_ref[0]
    t = tg_ref[0]
    m = jnp.max(x, axis=0)
    s = jnp.sum(jnp.exp(x - m[None]), axis=0)
    cls = lax.broadcasted_iota(jnp.int32, x.shape, 0)
    xt = jnp.sum(jnp.where(cls == t[None], x, 0.0), axis=0)
    loss = jnp.maximum(m + jnp.log(s) - xt, 0.0)
    out_f_ref[...] = loss.reshape(_BH * _W)
    bits = lax.bitcast_convert_type(loss, jnp.int32)
    out_i_ref[...] = ((bits & jnp.int32(-1024)) | t).reshape(_BH * _W)


def _ce_losses(logits, targets):
    flat = pl.BlockSpec((_BH * _W,), lambda b, h: (b * (_H // _BH) + h,))
    return pl.pallas_call(
        _ce_body,
        grid=(_B, _H // _BH),
        in_specs=[
            pl.BlockSpec((1, _C, _BH, _W), lambda b, h: (b, 0, h, 0)),
            pl.BlockSpec((1, _BH, _W), lambda b, h: (b, h, 0)),
        ],
        out_specs=[flat, flat],
        out_shape=[
            jax.ShapeDtypeStruct((_N,), jnp.float32),
            jax.ShapeDtypeStruct((_N,), jnp.int32),
        ],
    )(logits, targets)


_MESH = plsc.VectorSubcoreMesh(core_axis_name="c", subcore_axis_name="s")


def _wid():
    return lax.axis_index("s") * _NC + lax.axis_index("c")


def _zero(ref, n, dtype):
    z = jnp.zeros((_L,), dtype)

    @plsc.parallel_loop(0, n // _L, 1, unroll=8)
    def body(j):
        ref[pl.ds(j * _L, _L)] = z


def _reduce_rows(hist_hbm, buf_v, acc_v, nbins):
    rows = 8
    _zero(acc_v, nbins, jnp.int32)

    def chunk(ci, _):
        src = hist_hbm.at[pl.ds(pl.multiple_of(ci * (rows * nbins), 8),
                                rows * nbins)]
        pltpu.sync_copy(src, buf_v.at[pl.ds(0, rows * nbins)])

        @plsc.parallel_loop(0, nbins // _L, 1, unroll=4)
        def jbody(j):
            acc = acc_v[pl.ds(j * _L, _L)]
            for rr in range(rows):
                acc = acc + buf_v[pl.ds(rr * nbins + j * _L, _L)]
            acc_v[pl.ds(j * _L, _L)] = acc
        return 0

    lax.fori_loop(0, _NW // rows, chunk, 0)


def _find_kth(acc_v, nbins, kk):
    nch = nbins // _L
    lane = lax.iota(jnp.int32, _L)

    def body(i, carry):
        found, bfound, above, cum = carry
        j = nch - 1 - i
        v = acc_v[pl.ds(j * _L, _L)]
        rv = lax.rev(v, (0,))
        cs = plsc.cumsum(rv)
        tot = jnp.max(cs)
        hit = (cum + cs) >= kk
        anyhit = jnp.max(hit.astype(jnp.int32)) > 0
        ps = jnp.max(plsc.all_reduce_ffs(hit))
        bin_here = j * _L + (_L - 1) - ps
        above_here = cum + jnp.sum(jnp.where(lane < ps, rv, 0))
        take = jnp.logical_and(anyhit, found == 0)
        return (jnp.where(take, 1, found),
                jnp.where(take, bin_here, bfound),
                jnp.where(take, above_here, above),
                cum + tot)

    _, b, above, _ = lax.fori_loop(0, nch, body, (0, 0, 0, 0))
    return b, above


def _lane_reduce_store(tbl_v, red_v, nbins):
    @plsc.parallel_loop(0, nbins // _L, 1, unroll=4)
    def rbody(j):
        acc = tbl_v[pl.ds(j * _L, _L)]
        for l in range(1, _L):
            acc = acc + tbl_v[pl.ds(l * nbins + j * _L, _L)]
        red_v[pl.ds(j * _L, _L)] = acc


@functools.partial(
    pl.kernel,
    out_type=jax.ShapeDtypeStruct((_NW * _BA,), jnp.int32),
    mesh=_MESH,
    compiler_params=pltpu.CompilerParams(needs_layout_passes=False),
    scratch_types=[
        pltpu.VMEM((_PT,), jnp.int32),
        pltpu.VMEM((_L * _BA,), jnp.int32),
        pltpu.VMEM((_BA,), jnp.int32),
        pltpu.SemaphoreType.DMA,
    ],
)
def _hist_a(bits_hbm, out_hbm, data_v, tbl_v, red_v, dsem):
    wid = _wid()
    cp = pltpu.async_copy(
        bits_hbm.at[pl.ds(pl.multiple_of(wid * _PT, 8), _PT)], data_v, dsem)
    _zero(tbl_v, _L * _BA, jnp.int32)
    cp.wait()
    lb = lax.iota(jnp.int32, _L) * _BA
    ones = jnp.ones((_L,), jnp.int32)

    @plsc.parallel_loop(0, _PT // _L, 1, unroll=8)
    def body(i):
        bits = data_v[pl.ds(i * _L, _L)]
        plsc.addupdate_scatter(tbl_v, [lb + (bits >> 21)], ones)

    _lane_reduce_store(tbl_v, red_v, _BA)
    pltpu.sync_copy(red_v,
                    out_hbm.at[pl.ds(pl.multiple_of(wid * _BA, 8), _BA)])


@functools.partial(
    pl.kernel,
    out_type=jax.ShapeDtypeStruct((_NW * _BB,), jnp.int32),
    mesh=_MESH,
    compiler_params=pltpu.CompilerParams(needs_layout_passes=False),
    scratch_types=[
        pltpu.VMEM((_PT,), jnp.int32),
        pltpu.VMEM((_L * _BB,), jnp.int32),
        pltpu.VMEM((_BB,), jnp.int32),
        pltpu.VMEM((8 * _BA,), jnp.int32),
        pltpu.VMEM((_BA,), jnp.int32),
        pltpu.SemaphoreType.DMA,
    ],
)
def _hist_b(bits_hbm, ha_hbm, out_hbm, data_v, tbl_v, red_v, buf_v, acca_v,
            dsem):
    wid = _wid()
    cp = pltpu.async_copy(
        bits_hbm.at[pl.ds(pl.multiple_of(wid * _PT, 8), _PT)], data_v, dsem)
    _reduce_rows(ha_hbm, buf_v, acca_v, _BA)
    ba, _ = _find_kth(acca_v, _BA, _K)
    _zero(tbl_v, _L * _BB, jnp.int32)
    cp.wait()
    lb = lax.iota(jnp.int32, _L) * _BB
    ones = jnp.ones((_L,), jnp.int32)

    @plsc.parallel_loop(0, _PT // _L, 1, unroll=8)
    def body(i):
        bits = data_v[pl.ds(i * _L, _L)]
        m = (bits >> 21) == ba
        plsc.addupdate_scatter(tbl_v, [lb + ((bits >> 10) & 0x7FF)], ones,
                               mask=m)

    _lane_reduce_store(tbl_v, red_v, _BB)
    pltpu.sync_copy(red_v,
                    out_hbm.at[pl.ds(pl.multiple_of(wid * _BB, 8), _BB)])


_CH = 16384


@functools.partial(
    pl.kernel,
    out_type=jax.ShapeDtypeStruct((_NW * 128,), jnp.float32),
    mesh=_MESH,
    compiler_params=pltpu.CompilerParams(needs_layout_passes=False),
    scratch_types=[
        pltpu.VMEM((_CH,), jnp.int32),
        pltpu.VMEM((_CH,), jnp.int32),
        pltpu.VMEM((_CH,), jnp.float32),
        pltpu.VMEM((_CH,), jnp.float32),
        pltpu.VMEM((4 * _L * 32,), jnp.float32),
        pltpu.VMEM((128,), jnp.float32),
        pltpu.VMEM((8 * _BB,), jnp.int32),
        pltpu.VMEM((_BA,), jnp.int32),
        pltpu.VMEM((_BB,), jnp.int32),
        pltpu.SemaphoreType.DMA,
    ],
)
def _stats(loss_hbm, bits_hbm, ha_hbm, hb_hbm, out_hbm,
           bi0_v, bi1_v, lo0_v, lo1_v, tbl_v, stg_v, buf_v,
           acca_v, accb_v, sem):
    wid = _wid()
    nch = _PT // _CH
    bi = (bi0_v, bi1_v)
    lo = (lo0_v, lo1_v)

    def start(c, b):
        base = pl.multiple_of(wid * _PT + c * _CH, 8)
        return (pltpu.async_copy(bits_hbm.at[pl.ds(base, _CH)], bi[b], sem),
                pltpu.async_copy(loss_hbm.at[pl.ds(base, _CH)], lo[b], sem))

    hs = start(0, 0)
    _reduce_rows(ha_hbm, buf_v, acca_v, _BA)
    ba, above_a = _find_kth(acca_v, _BA, _K)
    _reduce_rows(hb_hbm, buf_v, accb_v, _BB)
    bb, _ = _find_kth(accb_v, _BB, _K - above_a)
    pfx = ba * _BB + bb

    _zero(tbl_v, 4 * _L * 32, jnp.float32)
    lane32 = lax.iota(jnp.int32, _L) * 32
    onesf = jnp.ones((_L,), jnp.float32)

    for c in range(nch):
        for h in hs:
            h.wait()
        if c + 1 < nch:
            hs = start(c + 1, (c + 1) % 2)
        bb_v, ll_v = bi[c % 2], lo[c % 2]

        @plsc.parallel_loop(0, _CH // _L, 1, unroll=8)
        def body(i, bb_v=bb_v, ll_v=ll_v):
            bits = bb_v[pl.ds(i * _L, _L)]
            v = ll_v[pl.ds(i * _L, _L)]
            t = bits & 31
            hi = bits >> 10
            mg = hi > pfx
            ma = hi >= pfx
            idx = lane32 + t
            off1 = jnp.where(mg, 0, 3 * _L * 32)
            off2 = jnp.where(mg, _L * 32, 2 * _L * 32)
            plsc.addupdate_scatter(tbl_v, [idx + off1], onesf, mask=ma)
            plsc.addupdate_scatter(tbl_v, [idx + off2], v, mask=ma)

    for r, so in ((0, 0), (1, 32), (3, 64), (2, 96)):
        def rbody(j, _, r=r, so=so):
            acc = tbl_v[pl.ds(r * _L * 32 + j * _L, _L)]
            for l in range(1, _L):
                acc = acc + tbl_v[pl.ds(r * _L * 32 + l * 32 + j * _L, _L)]
            stg_v[pl.ds(so + j * _L, _L)] = acc
            return 0

        lax.fori_loop(0, 2, rbody, 0)
    pltpu.sync_copy(stg_v,
                    out_hbm.at[pl.ds(pl.multiple_of(wid * 128, 8), 128)])


def _comb_body(st_ref, out_ref):
    x = st_ref[...]
    cnt = jnp.sum(x[:, 0:32], axis=0)
    s = jnp.sum(x[:, 32:64], axis=0)
    t = jnp.sum(x[:, 64:96], axis=0)
    stie = jnp.sum(x[:, 96:128], axis=0)
    r = _K - jnp.sum(cnt)
    ii = lax.broadcasted_iota(jnp.int32, (32, 32), 0)
    jj = lax.broadcasted_iota(jnp.int32, (32, 32), 1)
    pre = jnp.sum(jnp.where(ii < jj, t[:, None], 0.0), axis=0)
    a = jnp.clip(r - pre, 0.0, t)
    cnt_tot = cnt + a
    s_tot = s + a * (stie / jnp.maximum(t, 1.0))
    contrib = s_tot * lax.rsqrt(cnt_tot + 1e-8)
    out_ref[...] = jnp.reshape(jnp.sum(contrib) * (1.0 / _K ** 0.5), (1, 1))


def _combine(stats):
    return pl.pallas_call(
        _comb_body,
        in_specs=[pl.BlockSpec((_NW, 128), lambda: (0, 0))],
        out_shape=jax.ShapeDtypeStruct((1, 1), jnp.float32),
    )(stats)


def kernel(logits, targets):
    loss_f, loss_i = _ce_losses(logits, targets)
    ha = _hist_a(loss_i)
    hb = _hist_b(loss_i, ha)
    st = _stats(loss_f, loss_i, ha, hb)
    return _combine(st.reshape(_NW, 128)).reshape(())


# speedup vs baseline: 67.1294x; 1.0004x over previous
import functools

import jax
import jax.numpy as jnp
from jax import lax
from jax.experimental import pallas as pl
from jax.experimental.pallas import tpu as pltpu
from jax.experimental.pallas import tpu_sc as plsc

_C = 19
_B, _H, _W = 8, 512, 512
_N = _B * _H * _W
_K = int(0.2 * _N)
_NC, _NS, _L = 2, 16, 16
_NW = _NC * _NS
_PT = _N // _NW

_BA = 1024
_BB = 2048


_BH = 128


def _ce_body(lg_ref, tg_ref, out_f_ref, out_i_ref):
    x = lg_ref[0]
    t = tg_ref[0]
    m = jnp.max(x, axis=0)
    s = jnp.sum(jnp.exp(x - m[None]), axis=0)
    cls = lax.broadcasted_iota(jnp.int32, x.shape, 0)
    xt = jnp.sum(jnp.where(cls == t[None], x, 0.0), axis=0)
    loss = jnp.maximum(m + jnp.log(s) - xt, 0.0)
    out_f_ref[...] = loss.reshape(_BH * _W)
    bits = lax.bitcast_convert_type(loss, jnp.int32)
    out_i_ref[...] = ((bits & jnp.int32(-1024)) | t).reshape(_BH * _W)


def _ce_losses(logits, targets):
    flat = pl.BlockSpec((_BH * _W,), lambda b, h: (b * (_H // _BH) + h,))
    return pl.pallas_call(
        _ce_body,
        grid=(_B, _H // _BH),
        in_specs=[
            pl.BlockSpec((1, _C, _BH, _W), lambda b, h: (b, 0, h, 0)),
            pl.BlockSpec((1, _BH, _W), lambda b, h: (b, h, 0)),
        ],
        out_specs=[flat, flat],
        out_shape=[
            jax.ShapeDtypeStruct((_N,), jnp.float32),
            jax.ShapeDtypeStruct((_N,), jnp.int32),
        ],
    )(logits, targets)


_MESH = plsc.VectorSubcoreMesh(core_axis_name="c", subcore_axis_name="s")


def _wid():
    return lax.axis_index("s") * _NC + lax.axis_index("c")


def _zero(ref, n, dtype):
    z = jnp.zeros((_L,), dtype)

    @plsc.parallel_loop(0, n // _L, 1, unroll=8)
    def body(j):
        ref[pl.ds(j * _L, _L)] = z


def _reduce_rows(hist_hbm, buf_v, acc_v, nbins):
    rows = 8
    _zero(acc_v, nbins, jnp.int32)

    def chunk(ci, _):
        src = hist_hbm.at[pl.ds(pl.multiple_of(ci * (rows * nbins), 8),
                                rows * nbins)]
        pltpu.sync_copy(src, buf_v.at[pl.ds(0, rows * nbins)])

        @plsc.parallel_loop(0, nbins // _L, 1, unroll=4)
        def jbody(j):
            acc = acc_v[pl.ds(j * _L, _L)]
            for rr in range(rows):
                acc = acc + buf_v[pl.ds(rr * nbins + j * _L, _L)]
            acc_v[pl.ds(j * _L, _L)] = acc
        return 0

    lax.fori_loop(0, _NW // rows, chunk, 0)


def _find_kth(acc_v, nbins, kk):
    nch = nbins // _L
    lane = lax.iota(jnp.int32, _L)

    def body(i, carry):
        found, bfound, above, cum = carry
        j = nch - 1 - i
        v = acc_v[pl.ds(j * _L, _L)]
        rv = lax.rev(v, (0,))
        cs = plsc.cumsum(rv)
        tot = jnp.max(cs)
        hit = (cum + cs) >= kk
        anyhit = jnp.max(hit.astype(jnp.int32)) > 0
        ps = jnp.max(plsc.all_reduce_ffs(hit))
        bin_here = j * _L + (_L - 1) - ps
        above_here = cum + jnp.sum(jnp.where(lane < ps, rv, 0))
        take = jnp.logical_and(anyhit, found == 0)
        return (jnp.where(take, 1, found),
                jnp.where(take, bin_here, bfound),
                jnp.where(take, above_here, above),
                cum + tot)

    _, b, above, _ = lax.fori_loop(0, nch, body, (0, 0, 0, 0))
    return b, above


def _lane_reduce_store(tbl_v, red_v, nbins):
    @plsc.parallel_loop(0, nbins // _L, 1, unroll=4)
    def rbody(j):
        acc = tbl_v[pl.ds(j * _L, _L)]
        for l in range(1, _L):
            acc = acc + tbl_v[pl.ds(l * nbins + j * _L, _L)]
        red_v[pl.ds(j * _L, _L)] = acc


@functools.partial(
    pl.kernel,
    out_type=jax.ShapeDtypeStruct((_NW * _BA,), jnp.int32),
    mesh=_MESH,
    compiler_params=pltpu.CompilerParams(needs_layout_passes=False),
    scratch_types=[
        pltpu.VMEM((_PT,), jnp.int32),
        pltpu.VMEM((_L * _BA,), jnp.int32),
        pltpu.VMEM((_BA,), jnp.int32),
        pltpu.SemaphoreType.DMA,
    ],
)
def _hist_a(bits_hbm, out_hbm, data_v, tbl_v, red_v, dsem):
    wid = _wid()
    cp = pltpu.async_copy(
        bits_hbm.at[pl.ds(pl.multiple_of(wid * _PT, 8), _PT)], data_v, dsem)
    _zero(tbl_v, _L * _BA, jnp.int32)
    cp.wait()
    lb = lax.iota(jnp.int32, _L) * _BA
    ones = jnp.ones((_L,), jnp.int32)

    @plsc.parallel_loop(0, _PT // _L, 1, unroll=8)
    def body(i):
        bits = data_v[pl.ds(i * _L, _L)]
        plsc.addupdate_scatter(tbl_v, [lb + (bits >> 21)], ones)

    _lane_reduce_store(tbl_v, red_v, _BA)
    pltpu.sync_copy(red_v,
                    out_hbm.at[pl.ds(pl.multiple_of(wid * _BA, 8), _BA)])


@functools.partial(
    pl.kernel,
    out_type=jax.ShapeDtypeStruct((_NW * _BB,), jnp.int32),
    mesh=_MESH,
    compiler_params=pltpu.CompilerParams(needs_layout_passes=False),
    scratch_types=[
        pltpu.VMEM((_PT,), jnp.int32),
        pltpu.VMEM((_L * _BB,), jnp.int32),
        pltpu.VMEM((_BB,), jnp.int32),
        pltpu.VMEM((8 * _BA,), jnp.int32),
        pltpu.VMEM((_BA,), jnp.int32),
        pltpu.SemaphoreType.DMA,
    ],
)
def _hist_b(bits_hbm, ha_hbm, out_hbm, data_v, tbl_v, red_v, buf_v, acca_v,
            dsem):
    wid = _wid()
    cp = pltpu.async_copy(
        bits_hbm.at[pl.ds(pl.multiple_of(wid * _PT, 8), _PT)], data_v, dsem)
    _reduce_rows(ha_hbm, buf_v, acca_v, _BA)
    ba, _ = _find_kth(acca_v, _BA, _K)
    _zero(tbl_v, _L * _BB, jnp.int32)
    cp.wait()
    lb = lax.iota(jnp.int32, _L) * _BB
    ones = jnp.ones((_L,), jnp.int32)

    @plsc.parallel_loop(0, _PT // _L, 1, unroll=8)
    def body(i):
        bits = data_v[pl.ds(i * _L, _L)]
        m = (bits >> 21) == ba
        plsc.addupdate_scatter(tbl_v, [lb + ((bits >> 10) & 0x7FF)], ones,
                               mask=m)

    _lane_reduce_store(tbl_v, red_v, _BB)
    pltpu.sync_copy(red_v,
                    out_hbm.at[pl.ds(pl.multiple_of(wid * _BB, 8), _BB)])


_CH = 16384


@functools.partial(
    pl.kernel,
    out_type=jax.ShapeDtypeStruct((_NW * 128,), jnp.float32),
    mesh=_MESH,
    compiler_params=pltpu.CompilerParams(needs_layout_passes=False),
    scratch_types=[
        pltpu.VMEM((_CH,), jnp.int32),
        pltpu.VMEM((_CH,), jnp.int32),
        pltpu.VMEM((_CH,), jnp.float32),
        pltpu.VMEM((_CH,), jnp.float32),
        pltpu.VMEM((4 * _L * 32,), jnp.float32),
        pltpu.VMEM((128,), jnp.float32),
        pltpu.VMEM((8 * _BB,), jnp.int32),
        pltpu.VMEM((_BA,), jnp.int32),
        pltpu.VMEM((_BB,), jnp.int32),
        pltpu.SemaphoreType.DMA,
    ],
)
def _stats(loss_hbm, bits_hbm, ha_hbm, hb_hbm, out_hbm,
           bi0_v, bi1_v, lo0_v, lo1_v, tbl_v, stg_v, buf_v,
           acca_v, accb_v, sem):
    wid = _wid()
    nch = _PT // _CH
    bi = (bi0_v, bi1_v)
    lo = (lo0_v, lo1_v)

    def start(c, b):
        base = pl.multiple_of(wid * _PT + c * _CH, 8)
        return (pltpu.async_copy(bits_hbm.at[pl.ds(base, _CH)], bi[b], sem),
                pltpu.async_copy(loss_hbm.at[pl.ds(base, _CH)], lo[b], sem))

    hs = start(0, 0)
    _reduce_rows(ha_hbm, buf_v, acca_v, _BA)
    ba, above_a = _find_kth(acca_v, _BA, _K)
    _reduce_rows(hb_hbm, buf_v, accb_v, _BB)
    bb, _ = _find_kth(accb_v, _BB, _K - above_a)
    pfx = ba * _BB + bb

    _zero(tbl_v, 4 * _L * 32, jnp.float32)
    lane32 = lax.iota(jnp.int32, _L) * 32
    onesf = jnp.ones((_L,), jnp.float32)

    for c in range(nch):
        for h in hs:
            h.wait()
        if c + 1 < nch:
            hs = start(c + 1, (c + 1) % 2)
        bb_v, ll_v = bi[c % 2], lo[c % 2]

        @plsc.parallel_loop(0, _CH // _L, 1, unroll=8)
        def body(i, bb_v=bb_v, ll_v=ll_v):
            bits = bb_v[pl.ds(i * _L, _L)]
            v = ll_v[pl.ds(i * _L, _L)]
            t = bits & 31
            hi = bits >> 10
            mg = hi > pfx
            ma = hi >= pfx
            idx = lane32 + t
            off1 = jnp.where(mg, 0, 3 * _L * 32)
            off2 = jnp.where(mg, _L * 32, 2 * _L * 32)
            plsc.addupdate_scatter(tbl_v, [idx + off1], onesf, mask=ma)
            plsc.addupdate_scatter(tbl_v, [idx + off2], v, mask=ma)

    for r, so in ((0, 0), (1, 32), (3, 64), (2, 96)):
        def rbody(j, _, r=r, so=so):
            acc = tbl_v[pl.ds(r * _L * 32 + j * _L, _L)]
            for l in range(1, _L):
                acc = acc + tbl_v[pl.ds(r * _L * 32 + l * 32 + j * _L, _L)]
            stg_v[pl.ds(so + j * _L, _L)] = acc
            return 0

        lax.fori_loop(0, 2, rbody, 0)
    pltpu.sync_copy(stg_v,
                    out_hbm.at[pl.ds(pl.multiple_of(wid * 128, 8), 128)])


def _comb_body(st_ref, out_ref):
    x = st_ref[...]
    cnt = jnp.sum(x[:, 0:32], axis=0)
    s = jnp.sum(x[:, 32:64], axis=0)
    t = jnp.sum(x[:, 64:96], axis=0)
    stie = jnp.sum(x[:, 96:128], axis=0)
    r = _K - jnp.sum(cnt)
    ii = lax.broadcasted_iota(jnp.int32, (32, 32), 0)
    jj = lax.broadcasted_iota(jnp.int32, (32, 32), 1)
    pre = jnp.sum(jnp.where(ii < jj, t[:, None], 0.0), axis=0)
    a = jnp.clip(r - pre, 0.0, t)
    cnt_tot = cnt + a
    s_tot = s + a * (stie / jnp.maximum(t, 1.0))
    contrib = s_tot * lax.rsqrt(cnt_tot + 1e-8)
    out_ref[...] = jnp.reshape(jnp.sum(contrib) * (1.0 / _K ** 0.5), (1, 1))


def _combine(stats):
    return pl.pallas_call(
        _comb_body,
        in_specs=[pl.BlockSpec((_NW, 128), lambda: (0, 0))],
        out_shape=jax.ShapeDtypeStruct((1, 1), jnp.float32),
    )(stats)


def kernel(logits, targets):
    loss_f, loss_i = _ce_losses(logits, targets)
    ha = _hist_a(loss_i)
    hb = _hist_b(loss_i, ha)
    st = _stats(loss_f, loss_i, ha, hb)
    return _combine(st.reshape(_NW, 128)).reshape(())
